# Initial kernel scaffold; baseline (speedup 1.0000x reference)
import jax
import jax.numpy as jnp
from jax.experimental import pallas as pl


def kernel(f_atoms, f_bonds, a2b, b2a, b2revb, W_i_atom, W_i_bond, W_h_0, W_h_1, W_lr, W_o, b_o, gru_bias, gru_w_ih_f, gru_w_hh_f, gru_b_ih_f, gru_b_hh_f, gru_w_ih_b, gru_w_hh_b, gru_b_ih_b, gru_b_hh_b):
    raise NotImplementedError("write your pallas kernel here")


# speedup vs baseline: 1.9775x; 1.9775x over previous
import functools

import jax
import jax.numpy as jnp
from jax import lax
from jax.experimental import pallas as pl
from jax.experimental.pallas import tpu as pltpu
from jax.experimental.pallas import tpu_sc as plsc

_N_MOL = 16
_APM = 625
_NA = 1 + _N_MOL * _APM
_MAXNB = 16
_NB = 1 + _N_MOL * _APM * _MAXNB
_H = 256

_NW = 32
_LANES = 16

_A_CHUNK = 8
_A_CHUNKS_PW = 40
_A_PW = _A_CHUNK * _A_CHUNKS_PW
_A_PAD = _NW * _A_PW

_B_CHUNK = 128
_B_CHUNKS_PW = 40
_B_PW = _B_CHUNK * _B_CHUNKS_PW
_B_PAD = _NW * _B_PW

_BM = 512


def _sc_mesh():
    return plsc.VectorSubcoreMesh(core_axis_name="c", subcore_axis_name="s")


def _sc_agg(msg_bond, a2b_flat, base):
    add_base = base is not None
    n_idx = _A_CHUNK * _MAXNB

    def body(*refs):
        if add_base:
            (msg_ref, idx_ref, base_ref, out_ref,
             idx_v, rows_v, base_v, out_v, sem) = refs
        else:
            (msg_ref, idx_ref, out_ref,
             idx_v, rows_v, out_v, sem) = refs
        nc = lax.axis_size("c")
        wid = lax.axis_index("s") * nc + lax.axis_index("c")

        def chunk(ci, _):
            abase = wid * _A_PW + ci * _A_CHUNK
            pltpu.sync_copy(idx_ref.at[pl.ds(abase * _MAXNB, n_idx)], idx_v)
            pltpu.async_copy(msg_ref.at[idx_v], rows_v, sem).wait()
            if add_base:
                pltpu.sync_copy(base_ref.at[pl.ds(abase, _A_CHUNK)], base_v)

            def per_atom(a, _):
                r0 = a * _MAXNB
                for c in range(_H // _LANES):
                    sl = pl.ds(c * _LANES, _LANES)
                    v = [rows_v[r0 + j, sl] for j in range(_MAXNB)]
                    m = v[0]
                    for j in range(1, _MAXNB):
                        m = jnp.maximum(m, v[j])
                    n = _MAXNB
                    while n > 1:
                        h = n // 2
                        v = [v[j] + v[j + h] for j in range(h)]
                        n = h
                    res = v[0] * m
                    if add_base:
                        res = base_v[a, sl] + res
                    out_v[a, sl] = res
                return _

            lax.fori_loop(0, _A_CHUNK, per_atom, None)
            pltpu.sync_copy(out_v, out_ref.at[pl.ds(abase, _A_CHUNK)])
            return _

        lax.fori_loop(0, _A_CHUNKS_PW, chunk, None)

    scratch = [
        pltpu.VMEM((n_idx,), jnp.int32),
        pltpu.VMEM((n_idx, _H), jnp.float32),
    ]
    if add_base:
        scratch.append(pltpu.VMEM((_A_CHUNK, _H), jnp.float32))
    scratch += [
        pltpu.VMEM((_A_CHUNK, _H), jnp.float32),
        pltpu.SemaphoreType.DMA,
    ]
    k = pl.kernel(
        body,
        out_type=jax.ShapeDtypeStruct((_A_PAD, _H), jnp.float32),
        mesh=_sc_mesh(),
        scratch_types=scratch,
    )
    if add_base:
        return k(msg_bond, a2b_flat, base)
    return k(msg_bond, a2b_flat)


def _sc_bond(msg_atom, msg_bond, b2a, b2revb):
    def body(atom_ref, bond_ref, b2a_ref, b2revb_ref, out_ref,
             idxa_v, idxb_v, rows_a, rows_b, sem_a, sem_b):
        nc = lax.axis_size("c")
        wid = lax.axis_index("s") * nc + lax.axis_index("c")

        def chunk(ci, _):
            bbase = wid * _B_PW + ci * _B_CHUNK
            pltpu.sync_copy(b2a_ref.at[pl.ds(bbase, _B_CHUNK)], idxa_v)
            pltpu.sync_copy(b2revb_ref.at[pl.ds(bbase, _B_CHUNK)], idxb_v)
            ca = pltpu.async_copy(atom_ref.at[idxa_v], rows_a, sem_a)
            cb = pltpu.async_copy(bond_ref.at[idxb_v], rows_b, sem_b)
            ca.wait()
            cb.wait()

            def per_row(r, _):
                for c in range(_H // _LANES):
                    sl = pl.ds(c * _LANES, _LANES)
                    rows_a[r, sl] = rows_a[r, sl] - rows_b[r, sl]
                return _

            lax.fori_loop(0, _B_CHUNK, per_row, None)
            pltpu.sync_copy(rows_a, out_ref.at[pl.ds(bbase, _B_CHUNK)])
            return _

        lax.fori_loop(0, _B_CHUNKS_PW, chunk, None)

    return pl.kernel(
        body,
        out_type=jax.ShapeDtypeStruct((_B_PAD, _H), jnp.float32),
        mesh=_sc_mesh(),
        scratch_types=[
            pltpu.VMEM((_B_CHUNK,), jnp.int32),
            pltpu.VMEM((_B_CHUNK,), jnp.int32),
            pltpu.VMEM((_B_CHUNK, _H), jnp.float32),
            pltpu.VMEM((_B_CHUNK, _H), jnp.float32),
            pltpu.SemaphoreType.DMA,
            pltpu.SemaphoreType.DMA,
        ],
    )(msg_atom, msg_bond, b2a, b2revb)


def _tc_mm(x, w, add=None, relu=False, out_rows=None, grid_rows=None):
    m, kdim = x.shape
    n = w.shape[1]
    out_rows = out_rows or m
    grid_rows = grid_rows or m
    grid = pl.cdiv(grid_rows, _BM)

    def body(*refs):
        if add is not None:
            x_ref, w_ref, a_ref, o_ref = refs
        else:
            x_ref, w_ref, o_ref = refs
        acc = jnp.dot(x_ref[...], w_ref[...], preferred_element_type=jnp.float32)
        if add is not None:
            acc = acc + a_ref[...]
        if relu:
            acc = jnp.maximum(acc, 0.0)
        o_ref[...] = acc

    in_specs = [
        pl.BlockSpec((_BM, kdim), lambda i: (i, 0)),
        pl.BlockSpec((kdim, n), lambda i: (0, 0)),
    ]
    args = [x, w]
    if add is not None:
        in_specs.append(pl.BlockSpec((_BM, n), lambda i: (i, 0)))
        args.append(add)
    return pl.pallas_call(
        body,
        grid=(grid,),
        in_specs=in_specs,
        out_specs=pl.BlockSpec((_BM, n), lambda i: (i, 0)),
        out_shape=jax.ShapeDtypeStruct((out_rows, n), jnp.float32),
    )(*args)


def _tc_mm3(x1, w1, x2, w2, x3, w3):
    grid = _A_PAD // _BM

    def body(x1_ref, w1_ref, x2_ref, w2_ref, x3_ref, w3_ref, o_ref):
        acc = jnp.dot(x1_ref[...], w1_ref[...], preferred_element_type=jnp.float32)
        acc += jnp.dot(x2_ref[...], w2_ref[...], preferred_element_type=jnp.float32)
        acc += jnp.dot(x3_ref[...], w3_ref[...], preferred_element_type=jnp.float32)
        o_ref[...] = acc

    xspec = pl.BlockSpec((_BM, _H), lambda i: (i, 0))
    wspec = pl.BlockSpec((_H, _H), lambda i: (0, 0))
    return pl.pallas_call(
        body,
        grid=(grid,),
        in_specs=[xspec, wspec, xspec, wspec, xspec, wspec],
        out_specs=xspec,
        out_shape=jax.ShapeDtypeStruct((_A_PAD, _H), jnp.float32),
    )(x1, w1, x2, w2, x3, w3)


def _tc_h0(node_tm):
    tblk = 125
    grid = _APM // tblk

    def body(x_ref, o_ref):
        i = pl.program_id(0)
        bm = jnp.max(x_ref[...], axis=0)

        @pl.when(i == 0)
        def _init():
            o_ref[...] = bm

        @pl.when(i > 0)
        def _acc():
            o_ref[...] = jnp.maximum(o_ref[...], bm)

    return pl.pallas_call(
        body,
        grid=(grid,),
        in_specs=[pl.BlockSpec((tblk, _N_MOL, _H), lambda i: (i, 0, 0))],
        out_specs=pl.BlockSpec((_N_MOL, _H), lambda i: (0, 0)),
        out_shape=jax.ShapeDtypeStruct((_N_MOL, _H), jnp.float32),
    )(node_tm)


def _tc_gi(node_flat, gru_bias, wf_t, bf, wb_t, bb):
    m = node_flat.shape[0]
    grid = pl.cdiv(m, _BM)

    def body(x_ref, gbias_ref, wf_ref, bf_ref, wb_ref, bb_ref, of_ref, ob_ref):
        msg = jnp.maximum(x_ref[...] + gbias_ref[...], 0.0)
        of_ref[...] = jnp.dot(msg, wf_ref[...], preferred_element_type=jnp.float32) + bf_ref[...]
        ob_ref[...] = jnp.dot(msg, wb_ref[...], preferred_element_type=jnp.float32) + bb_ref[...]

    wspec = pl.BlockSpec((_H, 3 * _H), lambda i: (0, 0))
    bspec = pl.BlockSpec((1, 3 * _H), lambda i: (0, 0))
    ospec = pl.BlockSpec((_BM, 3 * _H), lambda i: (i, 0))
    oshape = jax.ShapeDtypeStruct((m, 3 * _H), jnp.float32)
    return pl.pallas_call(
        body,
        grid=(grid,),
        in_specs=[
            pl.BlockSpec((_BM, _H), lambda i: (i, 0)),
            pl.BlockSpec((1, _H), lambda i: (0, 0)),
            wspec, bspec, wspec, bspec,
        ],
        out_specs=[ospec, ospec],
        out_shape=[oshape, oshape],
    )(node_flat, gru_bias, wf_t, bf, wb_t, bb)


def _tc_gru(gi_f, gi_b, h0, whhf_t, whhb_t, bhf, bhb):
    T = _APM

    def body(gif_ref, gib_ref, h0_ref, wf_ref, wb_ref, bf_ref, bb_ref,
             of_ref, ob_ref, hf, hb):
        t = pl.program_id(0)

        @pl.when(t == 0)
        def _init():
            hf[...] = h0_ref[...]
            hb[...] = h0_ref[...]

        def step(gi, h, w_ref, b_ref):
            gh = jnp.dot(h, w_ref[...], preferred_element_type=jnp.float32) + b_ref[...]
            i_r = gi[:, :_H]
            i_z = gi[:, _H:2 * _H]
            i_n = gi[:, 2 * _H:]
            h_r = gh[:, :_H]
            h_z = gh[:, _H:2 * _H]
            h_n = gh[:, 2 * _H:]
            r = jax.nn.sigmoid(i_r + h_r)
            z = jax.nn.sigmoid(i_z + h_z)
            n = jnp.tanh(i_n + r * h_n)
            return (1.0 - z) * n + z * h

        hf_new = step(gif_ref[0], hf[...], wf_ref, bf_ref)
        hf[...] = hf_new
        of_ref[0] = hf_new

        hb_new = step(gib_ref[0], hb[...], wb_ref, bb_ref)
        hb[...] = hb_new
        ob_ref[0] = hb_new

    gspec_f = pl.BlockSpec((1, _N_MOL, 3 * _H), lambda t: (t, 0, 0))
    gspec_b = pl.BlockSpec((1, _N_MOL, 3 * _H), lambda t: (T - 1 - t, 0, 0))
    wspec = pl.BlockSpec((_H, 3 * _H), lambda t: (0, 0))
    bspec = pl.BlockSpec((1, 3 * _H), lambda t: (0, 0))
    ospec_f = pl.BlockSpec((1, _N_MOL, _H), lambda t: (t, 0, 0))
    ospec_b = pl.BlockSpec((1, _N_MOL, _H), lambda t: (T - 1 - t, 0, 0))
    oshape = jax.ShapeDtypeStruct((T, _N_MOL, _H), jnp.float32)
    return pl.pallas_call(
        body,
        grid=(T,),
        in_specs=[
            gspec_f, gspec_b,
            pl.BlockSpec((_N_MOL, _H), lambda t: (0, 0)),
            wspec, wspec, bspec, bspec,
        ],
        out_specs=[ospec_f, ospec_b],
        out_shape=[oshape, oshape],
        scratch_shapes=[
            pltpu.VMEM((_N_MOL, _H), jnp.float32),
            pltpu.VMEM((_N_MOL, _H), jnp.float32),
        ],
    )(gi_f, gi_b, h0, whhf_t, whhb_t, bhf, bhb)


def _tc_final(xf, xb, wof, wob, bo):
    tblk = 125
    rblk = tblk * _N_MOL
    grid = _APM // tblk

    def body(xf_ref, xb_ref, wof_ref, wob_ref, bo_ref, o_ref):
        i = pl.program_id(0)
        y = jnp.dot(xf_ref[...], wof_ref[...], preferred_element_type=jnp.float32)
        y += jnp.dot(xb_ref[...], wob_ref[...], preferred_element_type=jnp.float32)
        y = jnp.maximum(y + bo_ref[...], 0.0)
        part = jnp.sum(y.reshape(tblk, _N_MOL, _H), axis=0) * (1.0 / _APM)

        @pl.when(i == 0)
        def _init():
            o_ref[...] = part

        @pl.when(i > 0)
        def _acc():
            o_ref[...] = o_ref[...] + part

    xspec = pl.BlockSpec((rblk, _H), lambda i: (i, 0))
    wspec = pl.BlockSpec((_H, _H), lambda i: (0, 0))
    return pl.pallas_call(
        body,
        grid=(grid,),
        in_specs=[xspec, xspec, wspec, wspec,
                  pl.BlockSpec((1, _H), lambda i: (0, 0))],
        out_specs=pl.BlockSpec((_N_MOL, _H), lambda i: (0, 0)),
        out_shape=jax.ShapeDtypeStruct((_N_MOL, _H), jnp.float32),
    )(xf, xb, wof, wob, bo)


def kernel(f_atoms, f_bonds, a2b, b2a, b2revb, W_i_atom, W_i_bond, W_h_0,
           W_h_1, W_lr, W_o, b_o, gru_bias, gru_w_ih_f, gru_w_hh_f,
           gru_b_ih_f, gru_b_hh_f, gru_w_ih_b, gru_w_hh_b, gru_b_ih_b,
           gru_b_hh_b):
    i32 = jnp.int32
    a2b_flat = jnp.pad(a2b.astype(i32).reshape(-1), (0, (_A_PAD - _NA) * _MAXNB))
    b2a_p = jnp.pad(b2a.astype(i32), (0, _B_PAD - _NB))
    b2revb_p = jnp.pad(b2revb.astype(i32), (0, _B_PAD - _NB))

    grid_b = pl.cdiv(_NB, _BM) * _BM

    input_atom = _tc_mm(f_atoms, W_i_atom, relu=True,
                        out_rows=_A_PAD, grid_rows=_A_PAD)
    input_bond = _tc_mm(f_bonds, W_i_bond, relu=True,
                        out_rows=_B_PAD, grid_rows=grid_b)

    message_atom = input_atom
    message_bond = input_bond
    for W_h in (W_h_0, W_h_1):
        message_atom = _sc_agg(message_bond, a2b_flat, message_atom)
        tmp = _sc_bond(message_atom, message_bond, b2a_p, b2revb_p)
        message_bond = _tc_mm(tmp, W_h, add=input_bond, relu=True,
                              out_rows=_B_PAD, grid_rows=grid_b)

    agg = _sc_agg(message_bond, a2b_flat, None)
    node = _tc_mm3(agg, W_lr[:_H], message_atom, W_lr[_H:2 * _H],
                   input_atom, W_lr[2 * _H:])

    node_tm = node[1:_NA].reshape(_N_MOL, _APM, _H).transpose(1, 0, 2)
    h0 = _tc_h0(node_tm)
    gi_f, gi_b = _tc_gi(
        node_tm.reshape(_N_MOL * _APM, _H), gru_bias.reshape(1, _H),
        gru_w_ih_f.T, gru_b_ih_f.reshape(1, 3 * _H),
        gru_w_ih_b.T, gru_b_ih_b.reshape(1, 3 * _H))
    out_f, out_b = _tc_gru(
        gi_f.reshape(_APM, _N_MOL, 3 * _H), gi_b.reshape(_APM, _N_MOL, 3 * _H),
        h0, gru_w_hh_f.T, gru_w_hh_b.T,
        gru_b_hh_f.reshape(1, 3 * _H), gru_b_hh_b.reshape(1, 3 * _H))

    return _tc_final(out_f.reshape(-1, _H), out_b.reshape(-1, _H),
                     W_o[:_H], W_o[_H:], b_o.reshape(1, _H))


# speedup vs baseline: 2.3429x; 1.1848x over previous
import functools

import jax
import jax.numpy as jnp
from jax import lax
from jax.experimental import pallas as pl
from jax.experimental.pallas import tpu as pltpu
from jax.experimental.pallas import tpu_sc as plsc

_N_MOL = 16
_APM = 625
_NA = 1 + _N_MOL * _APM
_MAXNB = 16
_NB = 1 + _N_MOL * _APM * _MAXNB
_H = 256

_NW = 32
_LANES = 16

_A_CHUNK = 8
_A_CHUNKS_PW = 40
_A_PW = _A_CHUNK * _A_CHUNKS_PW
_A_PAD = _NW * _A_PW

_B_CHUNK = 64
_B_CHUNKS_PW = 80
_B_PW = _B_CHUNK * _B_CHUNKS_PW
_B_PAD = _NW * _B_PW

_BM = 512


def _sc_mesh():
    return plsc.VectorSubcoreMesh(core_axis_name="c", subcore_axis_name="s")


def _sc_agg(msg_bond, a2b_flat, base):
    add_base = base is not None
    n_idx = _A_CHUNK * _MAXNB
    n_ci = _A_CHUNKS_PW

    def body(*refs):
        if add_base:
            (msg_ref, idx_ref, base_ref, out_ref, idx_all, rows_v, base_v,
             out_v, sg0, sg1, sb0, sb1, so0, so1) = refs
        else:
            (msg_ref, idx_ref, out_ref, idx_all, rows_v,
             out_v, sg0, sg1, sb0, sb1, so0, so1) = refs
        sg = (sg0, sg1)
        sb = (sb0, sb1)
        so = (so0, so1)
        nc = lax.axis_size("c")
        wid = lax.axis_index("s") * nc + lax.axis_index("c")
        wbase = wid * _A_PW

        pltpu.sync_copy(idx_ref.at[pl.ds(wbase * _MAXNB, _A_PW * _MAXNB)],
                        idx_all)

        def start(ci, b):
            pltpu.async_copy(
                msg_ref.at[idx_all.at[pl.ds(ci * n_idx, n_idx)]],
                rows_v.at[b], sg[b])
            if add_base:
                pltpu.async_copy(
                    base_ref.at[pl.ds(wbase + ci * _A_CHUNK, _A_CHUNK)],
                    base_v.at[b], sb[b])

        def wait_in(b):
            pltpu.make_async_copy(msg_ref.at[pl.ds(0, n_idx)],
                                  rows_v.at[b], sg[b]).wait()
            if add_base:
                pltpu.make_async_copy(base_ref.at[pl.ds(0, _A_CHUNK)],
                                      base_v.at[b], sb[b]).wait()

        def compute(ci, b):
            def per_atom(a, _):
                r0 = a * _MAXNB
                for c in range(_H // _LANES):
                    sl = pl.ds(c * _LANES, _LANES)
                    v = [rows_v[b, r0 + j, sl] for j in range(_MAXNB)]
                    m = v[0]
                    for j in range(1, _MAXNB):
                        m = jnp.maximum(m, v[j])
                    n = _MAXNB
                    while n > 1:
                        h = n // 2
                        v = [v[j] + v[j + h] for j in range(h)]
                        n = h
                    res = v[0] * m
                    if add_base:
                        res = base_v[b, a, sl] + res
                    out_v[b, a, sl] = res
                return _

            lax.fori_loop(0, _A_CHUNK, per_atom, None)

        for b in range(2):
            start(b, b)

        def pair(p, _):
            for b in range(2):
                ci = p * 2 + b
                wait_in(b)

                @pl.when(p > 0)
                def _drain_out():
                    pltpu.make_async_copy(
                        out_v.at[b], out_ref.at[pl.ds(0, _A_CHUNK)],
                        so[b]).wait()

                compute(ci, b)
                pltpu.async_copy(out_v.at[b],
                                 out_ref.at[pl.ds(wbase + ci * _A_CHUNK,
                                                  _A_CHUNK)], so[b])

                @pl.when(p < n_ci // 2 - 1)
                def _prefetch():
                    start(ci + 2, b)
            return _

        lax.fori_loop(0, n_ci // 2, pair, None)
        for b in range(2):
            pltpu.make_async_copy(out_v.at[b], out_ref.at[pl.ds(0, _A_CHUNK)],
                                  so[b]).wait()

    scratch = [
        pltpu.VMEM((_A_PW * _MAXNB,), jnp.int32),
        pltpu.VMEM((2, n_idx, _H), jnp.float32),
    ]
    if add_base:
        scratch.append(pltpu.VMEM((2, _A_CHUNK, _H), jnp.float32))
    scratch += [
        pltpu.VMEM((2, _A_CHUNK, _H), jnp.float32),
    ] + [pltpu.SemaphoreType.DMA] * 6
    k = pl.kernel(
        body,
        out_type=jax.ShapeDtypeStruct((_A_PAD, _H), jnp.float32),
        mesh=_sc_mesh(),
        scratch_types=scratch,
    )
    if add_base:
        return k(msg_bond, a2b_flat, base)
    return k(msg_bond, a2b_flat)


def _sc_bond(msg_atom, msg_bond, b2a, b2revb):
    n_ci = _B_CHUNKS_PW

    def body(atom_ref, bond_ref, b2a_ref, b2revb_ref, out_ref,
             idxa_all, idxb_all, rows_a, rows_b, out_v,
             sa0, sa1, sb0, sb1, so0, so1):
        sa = (sa0, sa1)
        sb = (sb0, sb1)
        so = (so0, so1)
        nc = lax.axis_size("c")
        wid = lax.axis_index("s") * nc + lax.axis_index("c")
        wbase = wid * _B_PW

        pltpu.sync_copy(b2a_ref.at[pl.ds(wbase, _B_PW)], idxa_all)
        pltpu.sync_copy(b2revb_ref.at[pl.ds(wbase, _B_PW)], idxb_all)

        def start(ci, b):
            pltpu.async_copy(
                atom_ref.at[idxa_all.at[pl.ds(ci * _B_CHUNK, _B_CHUNK)]],
                rows_a.at[b], sa[b])
            pltpu.async_copy(
                bond_ref.at[idxb_all.at[pl.ds(ci * _B_CHUNK, _B_CHUNK)]],
                rows_b.at[b], sb[b])

        def wait_in(b):
            pltpu.make_async_copy(atom_ref.at[pl.ds(0, _B_CHUNK)],
                                  rows_a.at[b], sa[b]).wait()
            pltpu.make_async_copy(bond_ref.at[pl.ds(0, _B_CHUNK)],
                                  rows_b.at[b], sb[b]).wait()

        def compute(b):
            def per_row(r, _):
                for c in range(_H // _LANES):
                    sl = pl.ds(c * _LANES, _LANES)
                    out_v[b, r, sl] = rows_a[b, r, sl] - rows_b[b, r, sl]
                return _

            lax.fori_loop(0, _B_CHUNK, per_row, None)

        for b in range(2):
            start(b, b)

        def pair(p, _):
            for b in range(2):
                ci = p * 2 + b
                wait_in(b)

                @pl.when(p > 0)
                def _drain_out():
                    pltpu.make_async_copy(
                        out_v.at[b], out_ref.at[pl.ds(0, _B_CHUNK)],
                        so[b]).wait()

                compute(b)
                pltpu.async_copy(out_v.at[b],
                                 out_ref.at[pl.ds(wbase + ci * _B_CHUNK,
                                                  _B_CHUNK)], so[b])

                @pl.when(p < n_ci // 2 - 1)
                def _prefetch():
                    start(ci + 2, b)
            return _

        lax.fori_loop(0, n_ci // 2, pair, None)
        for b in range(2):
            pltpu.make_async_copy(out_v.at[b], out_ref.at[pl.ds(0, _B_CHUNK)],
                                  so[b]).wait()

    return pl.kernel(
        body,
        out_type=jax.ShapeDtypeStruct((_B_PAD, _H), jnp.float32),
        mesh=_sc_mesh(),
        scratch_types=[
            pltpu.VMEM((_B_PW,), jnp.int32),
            pltpu.VMEM((_B_PW,), jnp.int32),
            pltpu.VMEM((2, _B_CHUNK, _H), jnp.float32),
            pltpu.VMEM((2, _B_CHUNK, _H), jnp.float32),
            pltpu.VMEM((2, _B_CHUNK, _H), jnp.float32),
        ] + [pltpu.SemaphoreType.DMA] * 6,
    )(msg_atom, msg_bond, b2a, b2revb)


def _tc_mm(x, w, add=None, relu=False, out_rows=None, grid_rows=None):
    m, kdim = x.shape
    n = w.shape[1]
    out_rows = out_rows or m
    grid_rows = grid_rows or m
    grid = pl.cdiv(grid_rows, _BM)

    def body(*refs):
        if add is not None:
            x_ref, w_ref, a_ref, o_ref = refs
        else:
            x_ref, w_ref, o_ref = refs
        acc = jnp.dot(x_ref[...], w_ref[...], preferred_element_type=jnp.float32)
        if add is not None:
            acc = acc + a_ref[...]
        if relu:
            acc = jnp.maximum(acc, 0.0)
        o_ref[...] = acc

    in_specs = [
        pl.BlockSpec((_BM, kdim), lambda i: (i, 0)),
        pl.BlockSpec((kdim, n), lambda i: (0, 0)),
    ]
    args = [x, w]
    if add is not None:
        in_specs.append(pl.BlockSpec((_BM, n), lambda i: (i, 0)))
        args.append(add)
    return pl.pallas_call(
        body,
        grid=(grid,),
        in_specs=in_specs,
        out_specs=pl.BlockSpec((_BM, n), lambda i: (i, 0)),
        out_shape=jax.ShapeDtypeStruct((out_rows, n), jnp.float32),
    )(*args)


def _tc_mm3(x1, w1, x2, w2, x3, w3):
    grid = _A_PAD // _BM

    def body(x1_ref, w1_ref, x2_ref, w2_ref, x3_ref, w3_ref, o_ref):
        acc = jnp.dot(x1_ref[...], w1_ref[...], preferred_element_type=jnp.float32)
        acc += jnp.dot(x2_ref[...], w2_ref[...], preferred_element_type=jnp.float32)
        acc += jnp.dot(x3_ref[...], w3_ref[...], preferred_element_type=jnp.float32)
        o_ref[...] = acc

    xspec = pl.BlockSpec((_BM, _H), lambda i: (i, 0))
    wspec = pl.BlockSpec((_H, _H), lambda i: (0, 0))
    return pl.pallas_call(
        body,
        grid=(grid,),
        in_specs=[xspec, wspec, xspec, wspec, xspec, wspec],
        out_specs=xspec,
        out_shape=jax.ShapeDtypeStruct((_A_PAD, _H), jnp.float32),
    )(x1, w1, x2, w2, x3, w3)


def _tc_h0(node_tm):
    tblk = 125
    grid = _APM // tblk

    def body(x_ref, o_ref):
        i = pl.program_id(0)
        bm = jnp.max(x_ref[...], axis=0)

        @pl.when(i == 0)
        def _init():
            o_ref[...] = bm

        @pl.when(i > 0)
        def _acc():
            o_ref[...] = jnp.maximum(o_ref[...], bm)

    return pl.pallas_call(
        body,
        grid=(grid,),
        in_specs=[pl.BlockSpec((tblk, _N_MOL, _H), lambda i: (i, 0, 0))],
        out_specs=pl.BlockSpec((_N_MOL, _H), lambda i: (0, 0)),
        out_shape=jax.ShapeDtypeStruct((_N_MOL, _H), jnp.float32),
    )(node_tm)


def _tc_gi(node_flat, gru_bias, wf_t, bf, wb_t, bb):
    m = node_flat.shape[0]
    grid = pl.cdiv(m, _BM)

    def body(x_ref, gbias_ref, wf_ref, bf_ref, wb_ref, bb_ref, of_ref, ob_ref):
        msg = jnp.maximum(x_ref[...] + gbias_ref[...], 0.0)
        of_ref[...] = jnp.dot(msg, wf_ref[...], preferred_element_type=jnp.float32) + bf_ref[...]
        ob_ref[...] = jnp.dot(msg, wb_ref[...], preferred_element_type=jnp.float32) + bb_ref[...]

    wspec = pl.BlockSpec((_H, 3 * _H), lambda i: (0, 0))
    bspec = pl.BlockSpec((1, 3 * _H), lambda i: (0, 0))
    ospec = pl.BlockSpec((_BM, 3 * _H), lambda i: (i, 0))
    oshape = jax.ShapeDtypeStruct((m, 3 * _H), jnp.float32)
    return pl.pallas_call(
        body,
        grid=(grid,),
        in_specs=[
            pl.BlockSpec((_BM, _H), lambda i: (i, 0)),
            pl.BlockSpec((1, _H), lambda i: (0, 0)),
            wspec, bspec, wspec, bspec,
        ],
        out_specs=[ospec, ospec],
        out_shape=[oshape, oshape],
    )(node_flat, gru_bias, wf_t, bf, wb_t, bb)


def _tc_gru(gi_f, gi_b, h0, whhf_t, whhb_t, bhf, bhb):
    T = _APM

    def body(gif_ref, gib_ref, h0_ref, wf_ref, wb_ref, bf_ref, bb_ref,
             of_ref, ob_ref, hf, hb):
        t = pl.program_id(0)

        @pl.when(t == 0)
        def _init():
            hf[...] = h0_ref[...]
            hb[...] = h0_ref[...]

        def step(gi, h, w_ref, b_ref):
            gh = jnp.dot(h, w_ref[...], preferred_element_type=jnp.float32) + b_ref[...]
            i_r = gi[:, :_H]
            i_z = gi[:, _H:2 * _H]
            i_n = gi[:, 2 * _H:]
            h_r = gh[:, :_H]
            h_z = gh[:, _H:2 * _H]
            h_n = gh[:, 2 * _H:]
            r = jax.nn.sigmoid(i_r + h_r)
            z = jax.nn.sigmoid(i_z + h_z)
            n = jnp.tanh(i_n + r * h_n)
            return (1.0 - z) * n + z * h

        hf_new = step(gif_ref[0], hf[...], wf_ref, bf_ref)
        hf[...] = hf_new
        of_ref[0] = hf_new

        hb_new = step(gib_ref[0], hb[...], wb_ref, bb_ref)
        hb[...] = hb_new
        ob_ref[0] = hb_new

    gspec_f = pl.BlockSpec((1, _N_MOL, 3 * _H), lambda t: (t, 0, 0))
    gspec_b = pl.BlockSpec((1, _N_MOL, 3 * _H), lambda t: (T - 1 - t, 0, 0))
    wspec = pl.BlockSpec((_H, 3 * _H), lambda t: (0, 0))
    bspec = pl.BlockSpec((1, 3 * _H), lambda t: (0, 0))
    ospec_f = pl.BlockSpec((1, _N_MOL, _H), lambda t: (t, 0, 0))
    ospec_b = pl.BlockSpec((1, _N_MOL, _H), lambda t: (T - 1 - t, 0, 0))
    oshape = jax.ShapeDtypeStruct((T, _N_MOL, _H), jnp.float32)
    return pl.pallas_call(
        body,
        grid=(T,),
        in_specs=[
            gspec_f, gspec_b,
            pl.BlockSpec((_N_MOL, _H), lambda t: (0, 0)),
            wspec, wspec, bspec, bspec,
        ],
        out_specs=[ospec_f, ospec_b],
        out_shape=[oshape, oshape],
        scratch_shapes=[
            pltpu.VMEM((_N_MOL, _H), jnp.float32),
            pltpu.VMEM((_N_MOL, _H), jnp.float32),
        ],
    )(gi_f, gi_b, h0, whhf_t, whhb_t, bhf, bhb)


def _tc_final(xf, xb, wof, wob, bo):
    tblk = 125
    rblk = tblk * _N_MOL
    grid = _APM // tblk

    def body(xf_ref, xb_ref, wof_ref, wob_ref, bo_ref, o_ref):
        i = pl.program_id(0)
        y = jnp.dot(xf_ref[...], wof_ref[...], preferred_element_type=jnp.float32)
        y += jnp.dot(xb_ref[...], wob_ref[...], preferred_element_type=jnp.float32)
        y = jnp.maximum(y + bo_ref[...], 0.0)
        part = jnp.sum(y.reshape(tblk, _N_MOL, _H), axis=0) * (1.0 / _APM)

        @pl.when(i == 0)
        def _init():
            o_ref[...] = part

        @pl.when(i > 0)
        def _acc():
            o_ref[...] = o_ref[...] + part

    xspec = pl.BlockSpec((rblk, _H), lambda i: (i, 0))
    wspec = pl.BlockSpec((_H, _H), lambda i: (0, 0))
    return pl.pallas_call(
        body,
        grid=(grid,),
        in_specs=[xspec, xspec, wspec, wspec,
                  pl.BlockSpec((1, _H), lambda i: (0, 0))],
        out_specs=pl.BlockSpec((_N_MOL, _H), lambda i: (0, 0)),
        out_shape=jax.ShapeDtypeStruct((_N_MOL, _H), jnp.float32),
    )(xf, xb, wof, wob, bo)


def kernel(f_atoms, f_bonds, a2b, b2a, b2revb, W_i_atom, W_i_bond, W_h_0,
           W_h_1, W_lr, W_o, b_o, gru_bias, gru_w_ih_f, gru_w_hh_f,
           gru_b_ih_f, gru_b_hh_f, gru_w_ih_b, gru_w_hh_b, gru_b_ih_b,
           gru_b_hh_b):
    i32 = jnp.int32
    a2b_flat = jnp.pad(a2b.astype(i32).reshape(-1), (0, (_A_PAD - _NA) * _MAXNB))
    b2a_p = jnp.pad(b2a.astype(i32), (0, _B_PAD - _NB))
    b2revb_p = jnp.pad(b2revb.astype(i32), (0, _B_PAD - _NB))

    grid_b = pl.cdiv(_NB, _BM) * _BM

    input_atom = _tc_mm(f_atoms, W_i_atom, relu=True,
                        out_rows=_A_PAD, grid_rows=_A_PAD)
    input_bond = _tc_mm(f_bonds, W_i_bond, relu=True,
                        out_rows=_B_PAD, grid_rows=grid_b)

    message_atom = input_atom
    message_bond = input_bond
    for W_h in (W_h_0, W_h_1):
        message_atom = _sc_agg(message_bond, a2b_flat, message_atom)
        tmp = _sc_bond(message_atom, message_bond, b2a_p, b2revb_p)
        message_bond = _tc_mm(tmp, W_h, add=input_bond, relu=True,
                              out_rows=_B_PAD, grid_rows=grid_b)

    agg = _sc_agg(message_bond, a2b_flat, None)
    node = _tc_mm3(agg, W_lr[:_H], message_atom, W_lr[_H:2 * _H],
                   input_atom, W_lr[2 * _H:])

    node_tm = node[1:_NA].reshape(_N_MOL, _APM, _H).transpose(1, 0, 2)
    h0 = _tc_h0(node_tm)
    gi_f, gi_b = _tc_gi(
        node_tm.reshape(_N_MOL * _APM, _H), gru_bias.reshape(1, _H),
        gru_w_ih_f.T, gru_b_ih_f.reshape(1, 3 * _H),
        gru_w_ih_b.T, gru_b_ih_b.reshape(1, 3 * _H))
    out_f, out_b = _tc_gru(
        gi_f.reshape(_APM, _N_MOL, 3 * _H), gi_b.reshape(_APM, _N_MOL, 3 * _H),
        h0, gru_w_hh_f.T, gru_w_hh_b.T,
        gru_b_hh_f.reshape(1, 3 * _H), gru_b_hh_b.reshape(1, 3 * _H))

    return _tc_final(out_f.reshape(-1, _H), out_b.reshape(-1, _H),
                     W_o[:_H], W_o[_H:], b_o.reshape(1, _H))


# speedup vs baseline: 3.7152x; 1.5858x over previous
import functools

import jax
import jax.numpy as jnp
from jax import lax
from jax.experimental import pallas as pl
from jax.experimental.pallas import tpu as pltpu
from jax.experimental.pallas import tpu_sc as plsc

_N_MOL = 16
_APM = 625
_NA = 1 + _N_MOL * _APM
_MAXNB = 16
_NB = 1 + _N_MOL * _APM * _MAXNB
_H = 256

_NW = 32
_LANES = 16

_A_CHUNK = 8
_A_CHUNKS_PW = 40
_A_PW = _A_CHUNK * _A_CHUNKS_PW
_A_PAD = _NW * _A_PW

_B_CHUNK = 64
_B_CHUNKS_PW = 80
_B_PW = _B_CHUNK * _B_CHUNKS_PW
_B_PAD = _NW * _B_PW

_BM = 512


def _sc_mesh():
    return plsc.VectorSubcoreMesh(core_axis_name="c", subcore_axis_name="s")


def _sc_agg(msg_bond, a2b_flat, base):
    add_base = base is not None
    n_idx = _A_CHUNK * _MAXNB
    n_ci = _A_CHUNKS_PW

    def body(*refs):
        if add_base:
            (msg_ref, idx_ref, base_ref, out_ref, idx_all, rows_v, base_v,
             out_v, sg0, sg1, sb0, sb1, so0, so1) = refs
        else:
            (msg_ref, idx_ref, out_ref, idx_all, rows_v,
             out_v, sg0, sg1, sb0, sb1, so0, so1) = refs
        sg = (sg0, sg1)
        sb = (sb0, sb1)
        so = (so0, so1)
        nc = lax.axis_size("c")
        wid = lax.axis_index("s") * nc + lax.axis_index("c")
        wbase = wid * _A_PW

        pltpu.sync_copy(idx_ref.at[pl.ds(wbase * _MAXNB, _A_PW * _MAXNB)],
                        idx_all)

        def start(ci, b):
            pltpu.async_copy(
                msg_ref.at[idx_all.at[pl.ds(ci * n_idx, n_idx)]],
                rows_v.at[b], sg[b])
            if add_base:
                pltpu.async_copy(
                    base_ref.at[pl.ds(wbase + ci * _A_CHUNK, _A_CHUNK)],
                    base_v.at[b], sb[b])

        def wait_in(b):
            pltpu.make_async_copy(msg_ref.at[pl.ds(0, n_idx)],
                                  rows_v.at[b], sg[b]).wait()
            if add_base:
                pltpu.make_async_copy(base_ref.at[pl.ds(0, _A_CHUNK)],
                                      base_v.at[b], sb[b]).wait()

        def compute(ci, b):
            def per_atom(a, _):
                r0 = a * _MAXNB
                for c in range(_H // _LANES):
                    sl = pl.ds(c * _LANES, _LANES)
                    v = [rows_v[b, r0 + j, sl] for j in range(_MAXNB)]
                    m = v[0]
                    for j in range(1, _MAXNB):
                        m = jnp.maximum(m, v[j])
                    n = _MAXNB
                    while n > 1:
                        h = n // 2
                        v = [v[j] + v[j + h] for j in range(h)]
                        n = h
                    res = v[0] * m
                    if add_base:
                        res = base_v[b, a, sl] + res
                    out_v[b, a, sl] = res
                return _

            lax.fori_loop(0, _A_CHUNK, per_atom, None)

        for b in range(2):
            start(b, b)

        def pair(p, _):
            for b in range(2):
                ci = p * 2 + b
                wait_in(b)

                @pl.when(p > 0)
                def _drain_out():
                    pltpu.make_async_copy(
                        out_v.at[b], out_ref.at[pl.ds(0, _A_CHUNK)],
                        so[b]).wait()

                compute(ci, b)
                pltpu.async_copy(out_v.at[b],
                                 out_ref.at[pl.ds(wbase + ci * _A_CHUNK,
                                                  _A_CHUNK)], so[b])

                @pl.when(p < n_ci // 2 - 1)
                def _prefetch():
                    start(ci + 2, b)
            return _

        lax.fori_loop(0, n_ci // 2, pair, None)
        for b in range(2):
            pltpu.make_async_copy(out_v.at[b], out_ref.at[pl.ds(0, _A_CHUNK)],
                                  so[b]).wait()

    scratch = [
        pltpu.VMEM((_A_PW * _MAXNB,), jnp.int32),
        pltpu.VMEM((2, n_idx, _H), jnp.float32),
    ]
    if add_base:
        scratch.append(pltpu.VMEM((2, _A_CHUNK, _H), jnp.float32))
    scratch += [
        pltpu.VMEM((2, _A_CHUNK, _H), jnp.float32),
    ] + [pltpu.SemaphoreType.DMA] * 6
    k = pl.kernel(
        body,
        out_type=jax.ShapeDtypeStruct((_A_PAD, _H), jnp.float32),
        mesh=_sc_mesh(),
        scratch_types=scratch,
    )
    if add_base:
        return k(msg_bond, a2b_flat, base)
    return k(msg_bond, a2b_flat)


def _sc_bond(msg_atom, msg_bond, b2a, b2revb):
    n_ci = _B_CHUNKS_PW

    def body(atom_ref, bond_ref, b2a_ref, b2revb_ref, out_ref,
             idxa_all, idxb_all, rows_a, rows_b, out_v,
             sa0, sa1, sb0, sb1, so0, so1):
        sa = (sa0, sa1)
        sb = (sb0, sb1)
        so = (so0, so1)
        nc = lax.axis_size("c")
        wid = lax.axis_index("s") * nc + lax.axis_index("c")
        wbase = wid * _B_PW

        pltpu.sync_copy(b2a_ref.at[pl.ds(wbase, _B_PW)], idxa_all)
        pltpu.sync_copy(b2revb_ref.at[pl.ds(wbase, _B_PW)], idxb_all)

        def start(ci, b):
            pltpu.async_copy(
                atom_ref.at[idxa_all.at[pl.ds(ci * _B_CHUNK, _B_CHUNK)]],
                rows_a.at[b], sa[b])
            pltpu.async_copy(
                bond_ref.at[idxb_all.at[pl.ds(ci * _B_CHUNK, _B_CHUNK)]],
                rows_b.at[b], sb[b])

        def wait_in(b):
            pltpu.make_async_copy(atom_ref.at[pl.ds(0, _B_CHUNK)],
                                  rows_a.at[b], sa[b]).wait()
            pltpu.make_async_copy(bond_ref.at[pl.ds(0, _B_CHUNK)],
                                  rows_b.at[b], sb[b]).wait()

        def compute(b):
            def per_row(r, _):
                for c in range(_H // _LANES):
                    sl = pl.ds(c * _LANES, _LANES)
                    out_v[b, r, sl] = rows_a[b, r, sl] - rows_b[b, r, sl]
                return _

            lax.fori_loop(0, _B_CHUNK, per_row, None)

        for b in range(2):
            start(b, b)

        def pair(p, _):
            for b in range(2):
                ci = p * 2 + b
                wait_in(b)

                @pl.when(p > 0)
                def _drain_out():
                    pltpu.make_async_copy(
                        out_v.at[b], out_ref.at[pl.ds(0, _B_CHUNK)],
                        so[b]).wait()

                compute(b)
                pltpu.async_copy(out_v.at[b],
                                 out_ref.at[pl.ds(wbase + ci * _B_CHUNK,
                                                  _B_CHUNK)], so[b])

                @pl.when(p < n_ci // 2 - 1)
                def _prefetch():
                    start(ci + 2, b)
            return _

        lax.fori_loop(0, n_ci // 2, pair, None)
        for b in range(2):
            pltpu.make_async_copy(out_v.at[b], out_ref.at[pl.ds(0, _B_CHUNK)],
                                  so[b]).wait()

    return pl.kernel(
        body,
        out_type=jax.ShapeDtypeStruct((_B_PAD, _H), jnp.float32),
        mesh=_sc_mesh(),
        scratch_types=[
            pltpu.VMEM((_B_PW,), jnp.int32),
            pltpu.VMEM((_B_PW,), jnp.int32),
            pltpu.VMEM((2, _B_CHUNK, _H), jnp.float32),
            pltpu.VMEM((2, _B_CHUNK, _H), jnp.float32),
            pltpu.VMEM((2, _B_CHUNK, _H), jnp.float32),
        ] + [pltpu.SemaphoreType.DMA] * 6,
    )(msg_atom, msg_bond, b2a, b2revb)


def _tc_mm(x, w, add=None, relu=False, out_rows=None, grid_rows=None):
    m, kdim = x.shape
    n = w.shape[1]
    out_rows = out_rows or m
    grid_rows = grid_rows or m
    grid = pl.cdiv(grid_rows, _BM)

    def body(*refs):
        if add is not None:
            x_ref, w_ref, a_ref, o_ref = refs
        else:
            x_ref, w_ref, o_ref = refs
        acc = jnp.dot(x_ref[...], w_ref[...], preferred_element_type=jnp.float32)
        if add is not None:
            acc = acc + a_ref[...]
        if relu:
            acc = jnp.maximum(acc, 0.0)
        o_ref[...] = acc

    in_specs = [
        pl.BlockSpec((_BM, kdim), lambda i: (i, 0)),
        pl.BlockSpec((kdim, n), lambda i: (0, 0)),
    ]
    args = [x, w]
    if add is not None:
        in_specs.append(pl.BlockSpec((_BM, n), lambda i: (i, 0)))
        args.append(add)
    return pl.pallas_call(
        body,
        grid=(grid,),
        in_specs=in_specs,
        out_specs=pl.BlockSpec((_BM, n), lambda i: (i, 0)),
        out_shape=jax.ShapeDtypeStruct((out_rows, n), jnp.float32),
    )(*args)


def _tc_mm3(x1, w1, x2, w2, x3, w3):
    grid = _A_PAD // _BM

    def body(x1_ref, w1_ref, x2_ref, w2_ref, x3_ref, w3_ref, o_ref):
        acc = jnp.dot(x1_ref[...], w1_ref[...], preferred_element_type=jnp.float32)
        acc += jnp.dot(x2_ref[...], w2_ref[...], preferred_element_type=jnp.float32)
        acc += jnp.dot(x3_ref[...], w3_ref[...], preferred_element_type=jnp.float32)
        o_ref[...] = acc

    xspec = pl.BlockSpec((_BM, _H), lambda i: (i, 0))
    wspec = pl.BlockSpec((_H, _H), lambda i: (0, 0))
    return pl.pallas_call(
        body,
        grid=(grid,),
        in_specs=[xspec, wspec, xspec, wspec, xspec, wspec],
        out_specs=xspec,
        out_shape=jax.ShapeDtypeStruct((_A_PAD, _H), jnp.float32),
    )(x1, w1, x2, w2, x3, w3)


def _tc_h0(node_tm):
    tblk = 125
    grid = _APM // tblk

    def body(x_ref, o_ref):
        i = pl.program_id(0)
        bm = jnp.max(x_ref[...], axis=0)

        @pl.when(i == 0)
        def _init():
            o_ref[...] = bm

        @pl.when(i > 0)
        def _acc():
            o_ref[...] = jnp.maximum(o_ref[...], bm)

    return pl.pallas_call(
        body,
        grid=(grid,),
        in_specs=[pl.BlockSpec((tblk, _N_MOL, _H), lambda i: (i, 0, 0))],
        out_specs=pl.BlockSpec((_N_MOL, _H), lambda i: (0, 0)),
        out_shape=jax.ShapeDtypeStruct((_N_MOL, _H), jnp.float32),
    )(node_tm)


def _tc_gi(node_flat, gru_bias, wf_t, bf, wb_t, bb):
    m = node_flat.shape[0]
    grid = pl.cdiv(m, _BM)

    def body(x_ref, gbias_ref, wf_ref, bf_ref, wb_ref, bb_ref, of_ref, ob_ref):
        msg = jnp.maximum(x_ref[...] + gbias_ref[...], 0.0)
        of_ref[...] = jnp.dot(msg, wf_ref[...], preferred_element_type=jnp.float32) + bf_ref[...]
        ob_ref[...] = jnp.dot(msg, wb_ref[...], preferred_element_type=jnp.float32) + bb_ref[...]

    wspec = pl.BlockSpec((_H, 3 * _H), lambda i: (0, 0))
    bspec = pl.BlockSpec((1, 3 * _H), lambda i: (0, 0))
    ospec = pl.BlockSpec((_BM, 3 * _H), lambda i: (i, 0))
    oshape = jax.ShapeDtypeStruct((m, 3 * _H), jnp.float32)
    return pl.pallas_call(
        body,
        grid=(grid,),
        in_specs=[
            pl.BlockSpec((_BM, _H), lambda i: (i, 0)),
            pl.BlockSpec((1, _H), lambda i: (0, 0)),
            wspec, bspec, wspec, bspec,
        ],
        out_specs=[ospec, ospec],
        out_shape=[oshape, oshape],
    )(node_flat, gru_bias, wf_t, bf, wb_t, bb)


def _tc_gru(gi_f, gi_b, h0, whhf_t, whhb_t, bhf, bhb):
    T = _APM

    def body(gif_ref, gib_ref, h0_ref, wf_ref, wb_ref, bf_ref, bb_ref,
             of_ref, ob_ref, hf, hb):
        t = pl.program_id(0)

        @pl.when(t == 0)
        def _init():
            hf[...] = h0_ref[...]
            hb[...] = h0_ref[...]

        def step(gi, h, w_ref, b_ref):
            gh = jnp.dot(h, w_ref[...], preferred_element_type=jnp.float32) + b_ref[...]
            i_r = gi[:, :_H]
            i_z = gi[:, _H:2 * _H]
            i_n = gi[:, 2 * _H:]
            h_r = gh[:, :_H]
            h_z = gh[:, _H:2 * _H]
            h_n = gh[:, 2 * _H:]
            r = jax.nn.sigmoid(i_r + h_r)
            z = jax.nn.sigmoid(i_z + h_z)
            n = jnp.tanh(i_n + r * h_n)
            return (1.0 - z) * n + z * h

        hf_new = step(gif_ref[0], hf[...], wf_ref, bf_ref)
        hf[...] = hf_new
        of_ref[0] = hf_new

        hb_new = step(gib_ref[0], hb[...], wb_ref, bb_ref)
        hb[...] = hb_new
        ob_ref[0] = hb_new

    gspec_f = pl.BlockSpec((1, _N_MOL, 3 * _H), lambda t: (t, 0, 0))
    gspec_b = pl.BlockSpec((1, _N_MOL, 3 * _H), lambda t: (T - 1 - t, 0, 0))
    wspec = pl.BlockSpec((_H, 3 * _H), lambda t: (0, 0))
    bspec = pl.BlockSpec((1, 3 * _H), lambda t: (0, 0))
    ospec_f = pl.BlockSpec((1, _N_MOL, _H), lambda t: (t, 0, 0))
    ospec_b = pl.BlockSpec((1, _N_MOL, _H), lambda t: (T - 1 - t, 0, 0))
    oshape = jax.ShapeDtypeStruct((T, _N_MOL, _H), jnp.float32)
    return pl.pallas_call(
        body,
        grid=(T,),
        in_specs=[
            gspec_f, gspec_b,
            pl.BlockSpec((_N_MOL, _H), lambda t: (0, 0)),
            wspec, wspec, bspec, bspec,
        ],
        out_specs=[ospec_f, ospec_b],
        out_shape=[oshape, oshape],
        scratch_shapes=[
            pltpu.VMEM((_N_MOL, _H), jnp.float32),
            pltpu.VMEM((_N_MOL, _H), jnp.float32),
        ],
    )(gi_f, gi_b, h0, whhf_t, whhb_t, bhf, bhb)


def _tc_final(xf, xb, wof, wob, bo):
    tblk = 125
    rblk = tblk * _N_MOL
    grid = _APM // tblk

    def body(xf_ref, xb_ref, wof_ref, wob_ref, bo_ref, o_ref):
        i = pl.program_id(0)
        y = jnp.dot(xf_ref[...], wof_ref[...], preferred_element_type=jnp.float32)
        y += jnp.dot(xb_ref[...], wob_ref[...], preferred_element_type=jnp.float32)
        y = jnp.maximum(y + bo_ref[...], 0.0)
        part = jnp.sum(y.reshape(tblk, _N_MOL, _H), axis=0) * (1.0 / _APM)

        @pl.when(i == 0)
        def _init():
            o_ref[...] = part

        @pl.when(i > 0)
        def _acc():
            o_ref[...] = o_ref[...] + part

    xspec = pl.BlockSpec((rblk, _H), lambda i: (i, 0))
    wspec = pl.BlockSpec((_H, _H), lambda i: (0, 0))
    return pl.pallas_call(
        body,
        grid=(grid,),
        in_specs=[xspec, xspec, wspec, wspec,
                  pl.BlockSpec((1, _H), lambda i: (0, 0))],
        out_specs=pl.BlockSpec((_N_MOL, _H), lambda i: (0, 0)),
        out_shape=jax.ShapeDtypeStruct((_N_MOL, _H), jnp.float32),
    )(xf, xb, wof, wob, bo)


def kernel(f_atoms, f_bonds, a2b, b2a, b2revb, W_i_atom, W_i_bond, W_h_0,
           W_h_1, W_lr, W_o, b_o, gru_bias, gru_w_ih_f, gru_w_hh_f,
           gru_b_ih_f, gru_b_hh_f, gru_w_ih_b, gru_w_hh_b, gru_b_ih_b,
           gru_b_hh_b):
    i32 = jnp.int32
    pad_a = jnp.arange((_A_PAD - _NA) * _MAXNB, dtype=i32) * 37 % _NB
    a2b_flat = jnp.concatenate([a2b.astype(i32).reshape(-1), pad_a])
    pad_ba = jnp.arange(_B_PAD - _NB, dtype=i32) * 2 % _NA
    pad_br = jnp.arange(_B_PAD - _NB, dtype=i32) * 37 % _NB
    b2a_p = jnp.concatenate([b2a.astype(i32), pad_ba])
    b2revb_p = jnp.concatenate([b2revb.astype(i32), pad_br])

    grid_b = pl.cdiv(_NB, _BM) * _BM

    input_atom = _tc_mm(f_atoms, W_i_atom, relu=True,
                        out_rows=_A_PAD, grid_rows=_A_PAD)
    input_bond = _tc_mm(f_bonds, W_i_bond, relu=True,
                        out_rows=_B_PAD, grid_rows=grid_b)

    message_atom = input_atom
    message_bond = input_bond
    for W_h in (W_h_0, W_h_1):
        message_atom = _sc_agg(message_bond, a2b_flat, message_atom)
        tmp = _sc_bond(message_atom, message_bond, b2a_p, b2revb_p)
        message_bond = _tc_mm(tmp, W_h, add=input_bond, relu=True,
                              out_rows=_B_PAD, grid_rows=grid_b)

    agg = _sc_agg(message_bond, a2b_flat, None)
    node = _tc_mm3(agg, W_lr[:_H], message_atom, W_lr[_H:2 * _H],
                   input_atom, W_lr[2 * _H:])

    node_tm = node[1:_NA].reshape(_N_MOL, _APM, _H).transpose(1, 0, 2)
    h0 = _tc_h0(node_tm)
    gi_f, gi_b = _tc_gi(
        node_tm.reshape(_N_MOL * _APM, _H), gru_bias.reshape(1, _H),
        gru_w_ih_f.T, gru_b_ih_f.reshape(1, 3 * _H),
        gru_w_ih_b.T, gru_b_ih_b.reshape(1, 3 * _H))
    out_f, out_b = _tc_gru(
        gi_f.reshape(_APM, _N_MOL, 3 * _H), gi_b.reshape(_APM, _N_MOL, 3 * _H),
        h0, gru_w_hh_f.T, gru_w_hh_b.T,
        gru_b_hh_f.reshape(1, 3 * _H), gru_b_hh_b.reshape(1, 3 * _H))

    return _tc_final(out_f.reshape(-1, _H), out_b.reshape(-1, _H),
                     W_o[:_H], W_o[_H:], b_o.reshape(1, _H))


# speedup vs baseline: 4.1918x; 1.1283x over previous
import functools

import jax
import jax.numpy as jnp
from jax import lax
from jax.experimental import pallas as pl
from jax.experimental.pallas import tpu as pltpu
from jax.experimental.pallas import tpu_sc as plsc

_N_MOL = 16
_APM = 625
_NA = 1 + _N_MOL * _APM
_MAXNB = 16
_NB = 1 + _N_MOL * _APM * _MAXNB
_H = 256

_NW = 32
_LANES = 16

_A_CHUNK = 8
_A_CHUNKS_PW = 40
_A_PW = _A_CHUNK * _A_CHUNKS_PW
_A_PAD = _NW * _A_PW

_B_CHUNK = 64
_B_CHUNKS_PW = 80
_B_PW = _B_CHUNK * _B_CHUNKS_PW
_B_PAD = _NW * _B_PW

_BM = 512


def _sc_mesh():
    return plsc.VectorSubcoreMesh(core_axis_name="c", subcore_axis_name="s")


def _sc_agg(msg_bond, a2b_flat, base):
    add_base = base is not None
    n_idx = _A_CHUNK * _MAXNB
    n_ci = _A_CHUNKS_PW

    def body(*refs):
        if add_base:
            (msg_ref, idx_ref, base_ref, out_ref, idx_all, rows_v, base_v,
             out_v, sg0, sg1, sb0, sb1, so0, so1) = refs
        else:
            (msg_ref, idx_ref, out_ref, idx_all, rows_v,
             out_v, sg0, sg1, sb0, sb1, so0, so1) = refs
        sg = (sg0, sg1)
        sb = (sb0, sb1)
        so = (so0, so1)
        nc = lax.axis_size("c")
        wid = lax.axis_index("s") * nc + lax.axis_index("c")
        wbase = wid * _A_PW

        pltpu.sync_copy(idx_ref.at[pl.ds(wbase * _MAXNB, _A_PW * _MAXNB)],
                        idx_all)

        def start(ci, b):
            pltpu.async_copy(
                msg_ref.at[idx_all.at[pl.ds(ci * n_idx, n_idx)]],
                rows_v.at[b], sg[b])
            if add_base:
                pltpu.async_copy(
                    base_ref.at[pl.ds(wbase + ci * _A_CHUNK, _A_CHUNK)],
                    base_v.at[b], sb[b])

        def wait_in(b):
            pltpu.make_async_copy(msg_ref.at[pl.ds(0, n_idx)],
                                  rows_v.at[b], sg[b]).wait()
            if add_base:
                pltpu.make_async_copy(base_ref.at[pl.ds(0, _A_CHUNK)],
                                      base_v.at[b], sb[b]).wait()

        def compute(ci, b):
            def per_atom(a, _):
                r0 = a * _MAXNB
                for c in range(_H // _LANES):
                    sl = pl.ds(c * _LANES, _LANES)
                    v = [rows_v[b, r0 + j, sl] for j in range(_MAXNB)]
                    m = v[0]
                    for j in range(1, _MAXNB):
                        m = jnp.maximum(m, v[j])
                    n = _MAXNB
                    while n > 1:
                        h = n // 2
                        v = [v[j] + v[j + h] for j in range(h)]
                        n = h
                    res = v[0] * m
                    if add_base:
                        res = base_v[b, a, sl] + res
                    out_v[b, a, sl] = res
                return _

            lax.fori_loop(0, _A_CHUNK, per_atom, None)

        for b in range(2):
            start(b, b)

        def pair(p, _):
            for b in range(2):
                ci = p * 2 + b
                wait_in(b)

                @pl.when(p > 0)
                def _drain_out():
                    pltpu.make_async_copy(
                        out_v.at[b], out_ref.at[pl.ds(0, _A_CHUNK)],
                        so[b]).wait()

                compute(ci, b)
                pltpu.async_copy(out_v.at[b],
                                 out_ref.at[pl.ds(wbase + ci * _A_CHUNK,
                                                  _A_CHUNK)], so[b])

                @pl.when(p < n_ci // 2 - 1)
                def _prefetch():
                    start(ci + 2, b)
            return _

        lax.fori_loop(0, n_ci // 2, pair, None)
        for b in range(2):
            pltpu.make_async_copy(out_v.at[b], out_ref.at[pl.ds(0, _A_CHUNK)],
                                  so[b]).wait()

    scratch = [
        pltpu.VMEM((_A_PW * _MAXNB,), jnp.int32),
        pltpu.VMEM((2, n_idx, _H), jnp.float32),
    ]
    if add_base:
        scratch.append(pltpu.VMEM((2, _A_CHUNK, _H), jnp.float32))
    scratch += [
        pltpu.VMEM((2, _A_CHUNK, _H), jnp.float32),
    ] + [pltpu.SemaphoreType.DMA] * 6
    k = pl.kernel(
        body,
        out_type=jax.ShapeDtypeStruct((_A_PAD, _H), jnp.float32),
        mesh=_sc_mesh(),
        scratch_types=scratch,
    )
    if add_base:
        return k(msg_bond, a2b_flat, base)
    return k(msg_bond, a2b_flat)


def _sc_bond(msg_atom, msg_bond, b2a, b2revb):
    n_ci = _B_CHUNKS_PW

    def body(atom_ref, bond_ref, b2a_ref, b2revb_ref, out_ref,
             idxa_all, idxb_all, rows_a, rows_b, out_v,
             sa0, sa1, sb0, sb1, so0, so1):
        sa = (sa0, sa1)
        sb = (sb0, sb1)
        so = (so0, so1)
        nc = lax.axis_size("c")
        wid = lax.axis_index("s") * nc + lax.axis_index("c")
        wbase = wid * _B_PW

        pltpu.sync_copy(b2a_ref.at[pl.ds(wbase, _B_PW)], idxa_all)
        pltpu.sync_copy(b2revb_ref.at[pl.ds(wbase, _B_PW)], idxb_all)

        def start(ci, b):
            pltpu.async_copy(
                atom_ref.at[idxa_all.at[pl.ds(ci * _B_CHUNK, _B_CHUNK)]],
                rows_a.at[b], sa[b])
            pltpu.async_copy(
                bond_ref.at[idxb_all.at[pl.ds(ci * _B_CHUNK, _B_CHUNK)]],
                rows_b.at[b], sb[b])

        def wait_in(b):
            pltpu.make_async_copy(atom_ref.at[pl.ds(0, _B_CHUNK)],
                                  rows_a.at[b], sa[b]).wait()
            pltpu.make_async_copy(bond_ref.at[pl.ds(0, _B_CHUNK)],
                                  rows_b.at[b], sb[b]).wait()

        def compute(b):
            def per_row(r, _):
                for c in range(_H // _LANES):
                    sl = pl.ds(c * _LANES, _LANES)
                    out_v[b, r, sl] = rows_a[b, r, sl] - rows_b[b, r, sl]
                return _

            lax.fori_loop(0, _B_CHUNK, per_row, None)

        for b in range(2):
            start(b, b)

        def pair(p, _):
            for b in range(2):
                ci = p * 2 + b
                wait_in(b)

                @pl.when(p > 0)
                def _drain_out():
                    pltpu.make_async_copy(
                        out_v.at[b], out_ref.at[pl.ds(0, _B_CHUNK)],
                        so[b]).wait()

                compute(b)
                pltpu.async_copy(out_v.at[b],
                                 out_ref.at[pl.ds(wbase + ci * _B_CHUNK,
                                                  _B_CHUNK)], so[b])

                @pl.when(p < n_ci // 2 - 1)
                def _prefetch():
                    start(ci + 2, b)
            return _

        lax.fori_loop(0, n_ci // 2, pair, None)
        for b in range(2):
            pltpu.make_async_copy(out_v.at[b], out_ref.at[pl.ds(0, _B_CHUNK)],
                                  so[b]).wait()

    return pl.kernel(
        body,
        out_type=jax.ShapeDtypeStruct((_B_PAD, _H), jnp.float32),
        mesh=_sc_mesh(),
        scratch_types=[
            pltpu.VMEM((_B_PW,), jnp.int32),
            pltpu.VMEM((_B_PW,), jnp.int32),
            pltpu.VMEM((2, _B_CHUNK, _H), jnp.float32),
            pltpu.VMEM((2, _B_CHUNK, _H), jnp.float32),
            pltpu.VMEM((2, _B_CHUNK, _H), jnp.float32),
        ] + [pltpu.SemaphoreType.DMA] * 6,
    )(msg_atom, msg_bond, b2a, b2revb)


def _tc_mm(x, w, add=None, relu=False, out_rows=None, grid_rows=None):
    m, kdim = x.shape
    n = w.shape[1]
    out_rows = out_rows or m
    grid_rows = grid_rows or m
    grid = pl.cdiv(grid_rows, _BM)

    def body(*refs):
        if add is not None:
            x_ref, w_ref, a_ref, o_ref = refs
        else:
            x_ref, w_ref, o_ref = refs
        acc = jnp.dot(x_ref[...], w_ref[...], preferred_element_type=jnp.float32)
        if add is not None:
            acc = acc + a_ref[...]
        if relu:
            acc = jnp.maximum(acc, 0.0)
        o_ref[...] = acc

    in_specs = [
        pl.BlockSpec((_BM, kdim), lambda i: (i, 0)),
        pl.BlockSpec((kdim, n), lambda i: (0, 0)),
    ]
    args = [x, w]
    if add is not None:
        in_specs.append(pl.BlockSpec((_BM, n), lambda i: (i, 0)))
        args.append(add)
    return pl.pallas_call(
        body,
        grid=(grid,),
        in_specs=in_specs,
        out_specs=pl.BlockSpec((_BM, n), lambda i: (i, 0)),
        out_shape=jax.ShapeDtypeStruct((out_rows, n), jnp.float32),
    )(*args)


def _tc_mm3(x1, w1, x2, w2, x3, w3):
    grid = _A_PAD // _BM

    def body(x1_ref, w1_ref, x2_ref, w2_ref, x3_ref, w3_ref, o_ref):
        acc = jnp.dot(x1_ref[...], w1_ref[...], preferred_element_type=jnp.float32)
        acc += jnp.dot(x2_ref[...], w2_ref[...], preferred_element_type=jnp.float32)
        acc += jnp.dot(x3_ref[...], w3_ref[...], preferred_element_type=jnp.float32)
        o_ref[...] = acc

    xspec = pl.BlockSpec((_BM, _H), lambda i: (i, 0))
    wspec = pl.BlockSpec((_H, _H), lambda i: (0, 0))
    return pl.pallas_call(
        body,
        grid=(grid,),
        in_specs=[xspec, wspec, xspec, wspec, xspec, wspec],
        out_specs=xspec,
        out_shape=jax.ShapeDtypeStruct((_A_PAD, _H), jnp.float32),
    )(x1, w1, x2, w2, x3, w3)


def _tc_h0(node_tm):
    tblk = 125
    grid = _APM // tblk

    def body(x_ref, o_ref):
        i = pl.program_id(0)
        bm = jnp.max(x_ref[...], axis=0)

        @pl.when(i == 0)
        def _init():
            o_ref[...] = bm

        @pl.when(i > 0)
        def _acc():
            o_ref[...] = jnp.maximum(o_ref[...], bm)

    return pl.pallas_call(
        body,
        grid=(grid,),
        in_specs=[pl.BlockSpec((tblk, _N_MOL, _H), lambda i: (i, 0, 0))],
        out_specs=pl.BlockSpec((_N_MOL, _H), lambda i: (0, 0)),
        out_shape=jax.ShapeDtypeStruct((_N_MOL, _H), jnp.float32),
    )(node_tm)


def _tc_gi(node_flat, gru_bias, wf_t, bf, wb_t, bb):
    m = node_flat.shape[0]
    grid = pl.cdiv(m, _BM)

    def body(x_ref, gbias_ref, wf_ref, bf_ref, wb_ref, bb_ref, of_ref, ob_ref):
        msg = jnp.maximum(x_ref[...] + gbias_ref[...], 0.0)
        of_ref[...] = jnp.dot(msg, wf_ref[...], preferred_element_type=jnp.float32) + bf_ref[...]
        ob_ref[...] = jnp.dot(msg, wb_ref[...], preferred_element_type=jnp.float32) + bb_ref[...]

    wspec = pl.BlockSpec((_H, 3 * _H), lambda i: (0, 0))
    bspec = pl.BlockSpec((1, 3 * _H), lambda i: (0, 0))
    ospec = pl.BlockSpec((_BM, 3 * _H), lambda i: (i, 0))
    oshape = jax.ShapeDtypeStruct((m, 3 * _H), jnp.float32)
    return pl.pallas_call(
        body,
        grid=(grid,),
        in_specs=[
            pl.BlockSpec((_BM, _H), lambda i: (i, 0)),
            pl.BlockSpec((1, _H), lambda i: (0, 0)),
            wspec, bspec, wspec, bspec,
        ],
        out_specs=[ospec, ospec],
        out_shape=[oshape, oshape],
    )(node_flat, gru_bias, wf_t, bf, wb_t, bb)


def _tc_gru(gi_f, gi_b, h0, whhf_t, whhb_t, bhf, bhb):
    T = _APM
    TSUB = 5
    NBLK = T // TSUB

    def body(gif_ref, gib_ref, h0_ref, wf_ref, wb_ref, bf_ref, bb_ref,
             of_ref, ob_ref, hf, hb):
        t = pl.program_id(0)

        @pl.when(t == 0)
        def _init():
            hf[...] = h0_ref[...]
            hb[...] = h0_ref[...]

        def step(gi, h, w_ref, b_ref):
            gh = jnp.dot(h, w_ref[...], preferred_element_type=jnp.float32) + b_ref[...]
            i_r = gi[:, :_H]
            i_z = gi[:, _H:2 * _H]
            i_n = gi[:, 2 * _H:]
            h_r = gh[:, :_H]
            h_z = gh[:, _H:2 * _H]
            h_n = gh[:, 2 * _H:]
            r = jax.nn.sigmoid(i_r + h_r)
            z = jax.nn.sigmoid(i_z + h_z)
            n = jnp.tanh(i_n + r * h_n)
            return (1.0 - z) * n + z * h

        hfv = hf[...]
        hbv = hb[...]
        for k in range(TSUB):
            hfv = step(gif_ref[k], hfv, wf_ref, bf_ref)
            of_ref[k] = hfv
            kb = TSUB - 1 - k
            hbv = step(gib_ref[kb], hbv, wb_ref, bb_ref)
            ob_ref[kb] = hbv
        hf[...] = hfv
        hb[...] = hbv

    gspec_f = pl.BlockSpec((TSUB, _N_MOL, 3 * _H), lambda t: (t, 0, 0))
    gspec_b = pl.BlockSpec((TSUB, _N_MOL, 3 * _H), lambda t: (NBLK - 1 - t, 0, 0))
    wspec = pl.BlockSpec((_H, 3 * _H), lambda t: (0, 0))
    bspec = pl.BlockSpec((1, 3 * _H), lambda t: (0, 0))
    ospec_f = pl.BlockSpec((TSUB, _N_MOL, _H), lambda t: (t, 0, 0))
    ospec_b = pl.BlockSpec((TSUB, _N_MOL, _H), lambda t: (NBLK - 1 - t, 0, 0))
    oshape = jax.ShapeDtypeStruct((T, _N_MOL, _H), jnp.float32)
    return pl.pallas_call(
        body,
        grid=(NBLK,),
        in_specs=[
            gspec_f, gspec_b,
            pl.BlockSpec((_N_MOL, _H), lambda t: (0, 0)),
            wspec, wspec, bspec, bspec,
        ],
        out_specs=[ospec_f, ospec_b],
        out_shape=[oshape, oshape],
        scratch_shapes=[
            pltpu.VMEM((_N_MOL, _H), jnp.float32),
            pltpu.VMEM((_N_MOL, _H), jnp.float32),
        ],
    )(gi_f, gi_b, h0, whhf_t, whhb_t, bhf, bhb)


def _tc_final(xf, xb, wof, wob, bo):
    tblk = 125
    rblk = tblk * _N_MOL
    grid = _APM // tblk

    def body(xf_ref, xb_ref, wof_ref, wob_ref, bo_ref, o_ref):
        i = pl.program_id(0)
        y = jnp.dot(xf_ref[...], wof_ref[...], preferred_element_type=jnp.float32)
        y += jnp.dot(xb_ref[...], wob_ref[...], preferred_element_type=jnp.float32)
        y = jnp.maximum(y + bo_ref[...], 0.0)
        part = jnp.sum(y.reshape(tblk, _N_MOL, _H), axis=0) * (1.0 / _APM)

        @pl.when(i == 0)
        def _init():
            o_ref[...] = part

        @pl.when(i > 0)
        def _acc():
            o_ref[...] = o_ref[...] + part

    xspec = pl.BlockSpec((rblk, _H), lambda i: (i, 0))
    wspec = pl.BlockSpec((_H, _H), lambda i: (0, 0))
    return pl.pallas_call(
        body,
        grid=(grid,),
        in_specs=[xspec, xspec, wspec, wspec,
                  pl.BlockSpec((1, _H), lambda i: (0, 0))],
        out_specs=pl.BlockSpec((_N_MOL, _H), lambda i: (0, 0)),
        out_shape=jax.ShapeDtypeStruct((_N_MOL, _H), jnp.float32),
    )(xf, xb, wof, wob, bo)


def kernel(f_atoms, f_bonds, a2b, b2a, b2revb, W_i_atom, W_i_bond, W_h_0,
           W_h_1, W_lr, W_o, b_o, gru_bias, gru_w_ih_f, gru_w_hh_f,
           gru_b_ih_f, gru_b_hh_f, gru_w_ih_b, gru_w_hh_b, gru_b_ih_b,
           gru_b_hh_b):
    i32 = jnp.int32
    pad_a = jnp.arange((_A_PAD - _NA) * _MAXNB, dtype=i32) * 37 % _NB
    a2b_flat = jnp.concatenate([a2b.astype(i32).reshape(-1), pad_a])
    pad_ba = jnp.arange(_B_PAD - _NB, dtype=i32) * 2 % _NA
    pad_br = jnp.arange(_B_PAD - _NB, dtype=i32) * 37 % _NB
    b2a_p = jnp.concatenate([b2a.astype(i32), pad_ba])
    b2revb_p = jnp.concatenate([b2revb.astype(i32), pad_br])

    grid_b = pl.cdiv(_NB, _BM) * _BM

    input_atom = _tc_mm(f_atoms, W_i_atom, relu=True,
                        out_rows=_A_PAD, grid_rows=_A_PAD)
    input_bond = _tc_mm(f_bonds, W_i_bond, relu=True,
                        out_rows=_B_PAD, grid_rows=grid_b)

    message_atom = input_atom
    message_bond = input_bond
    for W_h in (W_h_0, W_h_1):
        message_atom = _sc_agg(message_bond, a2b_flat, message_atom)
        tmp = _sc_bond(message_atom, message_bond, b2a_p, b2revb_p)
        message_bond = _tc_mm(tmp, W_h, add=input_bond, relu=True,
                              out_rows=_B_PAD, grid_rows=grid_b)

    agg = _sc_agg(message_bond, a2b_flat, None)
    node = _tc_mm3(agg, W_lr[:_H], message_atom, W_lr[_H:2 * _H],
                   input_atom, W_lr[2 * _H:])

    node_tm = node[1:_NA].reshape(_N_MOL, _APM, _H).transpose(1, 0, 2)
    h0 = _tc_h0(node_tm)
    gi_f, gi_b = _tc_gi(
        node_tm.reshape(_N_MOL * _APM, _H), gru_bias.reshape(1, _H),
        gru_w_ih_f.T, gru_b_ih_f.reshape(1, 3 * _H),
        gru_w_ih_b.T, gru_b_ih_b.reshape(1, 3 * _H))
    out_f, out_b = _tc_gru(
        gi_f.reshape(_APM, _N_MOL, 3 * _H), gi_b.reshape(_APM, _N_MOL, 3 * _H),
        h0, gru_w_hh_f.T, gru_w_hh_b.T,
        gru_b_hh_f.reshape(1, 3 * _H), gru_b_hh_b.reshape(1, 3 * _H))

    return _tc_final(out_f.reshape(-1, _H), out_b.reshape(-1, _H),
                     W_o[:_H], W_o[_H:], b_o.reshape(1, _H))


# speedup vs baseline: 4.2555x; 1.0152x over previous
import functools

import jax
import jax.numpy as jnp
from jax import lax
from jax.experimental import pallas as pl
from jax.experimental.pallas import tpu as pltpu
from jax.experimental.pallas import tpu_sc as plsc

_N_MOL = 16
_APM = 625
_NA = 1 + _N_MOL * _APM
_MAXNB = 16
_NB = 1 + _N_MOL * _APM * _MAXNB
_H = 256

_NW = 32
_LANES = 16

_A_CHUNK = 8
_A_CHUNKS_PW = 40
_A_PW = _A_CHUNK * _A_CHUNKS_PW
_A_PAD = _NW * _A_PW

_B_CHUNK = 64
_B_CHUNKS_PW = 80
_B_PW = _B_CHUNK * _B_CHUNKS_PW
_B_PAD = _NW * _B_PW

_BM = 512


def _sc_mesh():
    return plsc.VectorSubcoreMesh(core_axis_name="c", subcore_axis_name="s")


def _sc_agg(msg_bond, a2b_flat, base):
    add_base = base is not None
    n_idx = _A_CHUNK * _MAXNB
    n_ci = _A_CHUNKS_PW

    def body(*refs):
        if add_base:
            (msg_ref, idx_ref, base_ref, out_ref, idx_all, rows_v, base_v,
             out_v, sg0, sg1, sb0, sb1, so0, so1) = refs
        else:
            (msg_ref, idx_ref, out_ref, idx_all, rows_v,
             out_v, sg0, sg1, sb0, sb1, so0, so1) = refs
        sg = (sg0, sg1)
        sb = (sb0, sb1)
        so = (so0, so1)
        nc = lax.axis_size("c")
        wid = lax.axis_index("s") * nc + lax.axis_index("c")
        wbase = wid * _A_PW

        pltpu.sync_copy(idx_ref.at[pl.ds(wbase * _MAXNB, _A_PW * _MAXNB)],
                        idx_all)

        def start(ci, b):
            pltpu.async_copy(
                msg_ref.at[idx_all.at[pl.ds(ci * n_idx, n_idx)]],
                rows_v.at[b], sg[b])
            if add_base:
                pltpu.async_copy(
                    base_ref.at[pl.ds(wbase + ci * _A_CHUNK, _A_CHUNK)],
                    base_v.at[b], sb[b])

        def wait_in(b):
            pltpu.make_async_copy(msg_ref.at[pl.ds(0, n_idx)],
                                  rows_v.at[b], sg[b]).wait()
            if add_base:
                pltpu.make_async_copy(base_ref.at[pl.ds(0, _A_CHUNK)],
                                      base_v.at[b], sb[b]).wait()

        def compute(ci, b):
            def per_atom(a, _):
                r0 = a * _MAXNB
                for c in range(_H // _LANES):
                    sl = pl.ds(c * _LANES, _LANES)
                    v = [rows_v[b, r0 + j, sl] for j in range(_MAXNB)]
                    m = v[0]
                    for j in range(1, _MAXNB):
                        m = jnp.maximum(m, v[j])
                    n = _MAXNB
                    while n > 1:
                        h = n // 2
                        v = [v[j] + v[j + h] for j in range(h)]
                        n = h
                    res = v[0] * m
                    if add_base:
                        res = base_v[b, a, sl] + res
                    out_v[b, a, sl] = res
                return _

            lax.fori_loop(0, _A_CHUNK, per_atom, None)

        for b in range(2):
            start(b, b)

        def pair(p, _):
            for b in range(2):
                ci = p * 2 + b
                wait_in(b)

                @pl.when(p > 0)
                def _drain_out():
                    pltpu.make_async_copy(
                        out_v.at[b], out_ref.at[pl.ds(0, _A_CHUNK)],
                        so[b]).wait()

                compute(ci, b)
                pltpu.async_copy(out_v.at[b],
                                 out_ref.at[pl.ds(wbase + ci * _A_CHUNK,
                                                  _A_CHUNK)], so[b])

                @pl.when(p < n_ci // 2 - 1)
                def _prefetch():
                    start(ci + 2, b)
            return _

        lax.fori_loop(0, n_ci // 2, pair, None)
        for b in range(2):
            pltpu.make_async_copy(out_v.at[b], out_ref.at[pl.ds(0, _A_CHUNK)],
                                  so[b]).wait()

    scratch = [
        pltpu.VMEM((_A_PW * _MAXNB,), jnp.int32),
        pltpu.VMEM((2, n_idx, _H), jnp.float32),
    ]
    if add_base:
        scratch.append(pltpu.VMEM((2, _A_CHUNK, _H), jnp.float32))
    scratch += [
        pltpu.VMEM((2, _A_CHUNK, _H), jnp.float32),
    ] + [pltpu.SemaphoreType.DMA] * 6
    k = pl.kernel(
        body,
        out_type=jax.ShapeDtypeStruct((_A_PAD, _H), jnp.float32),
        mesh=_sc_mesh(),
        scratch_types=scratch,
    )
    if add_base:
        return k(msg_bond, a2b_flat, base)
    return k(msg_bond, a2b_flat)


def _sc_bond(msg_atom, msg_bond, b2a, b2revb):
    n_ci = _B_CHUNKS_PW

    def body(atom_ref, bond_ref, b2a_ref, b2revb_ref, out_ref,
             idxa_all, idxb_all, rows_a, rows_b, out_v,
             sa0, sa1, sb0, sb1, so0, so1):
        sa = (sa0, sa1)
        sb = (sb0, sb1)
        so = (so0, so1)
        nc = lax.axis_size("c")
        wid = lax.axis_index("s") * nc + lax.axis_index("c")
        wbase = wid * _B_PW

        pltpu.sync_copy(b2a_ref.at[pl.ds(wbase, _B_PW)], idxa_all)
        pltpu.sync_copy(b2revb_ref.at[pl.ds(wbase, _B_PW)], idxb_all)

        def start(ci, b):
            pltpu.async_copy(
                atom_ref.at[idxa_all.at[pl.ds(ci * _B_CHUNK, _B_CHUNK)]],
                rows_a.at[b], sa[b])
            pltpu.async_copy(
                bond_ref.at[idxb_all.at[pl.ds(ci * _B_CHUNK, _B_CHUNK)]],
                rows_b.at[b], sb[b])

        def wait_in(b):
            pltpu.make_async_copy(atom_ref.at[pl.ds(0, _B_CHUNK)],
                                  rows_a.at[b], sa[b]).wait()
            pltpu.make_async_copy(bond_ref.at[pl.ds(0, _B_CHUNK)],
                                  rows_b.at[b], sb[b]).wait()

        def compute(b):
            def per_row(r, _):
                for c in range(_H // _LANES):
                    sl = pl.ds(c * _LANES, _LANES)
                    out_v[b, r, sl] = rows_a[b, r, sl] - rows_b[b, r, sl]
                return _

            lax.fori_loop(0, _B_CHUNK, per_row, None)

        for b in range(2):
            start(b, b)

        def pair(p, _):
            for b in range(2):
                ci = p * 2 + b
                wait_in(b)

                @pl.when(p > 0)
                def _drain_out():
                    pltpu.make_async_copy(
                        out_v.at[b], out_ref.at[pl.ds(0, _B_CHUNK)],
                        so[b]).wait()

                compute(b)
                pltpu.async_copy(out_v.at[b],
                                 out_ref.at[pl.ds(wbase + ci * _B_CHUNK,
                                                  _B_CHUNK)], so[b])

                @pl.when(p < n_ci // 2 - 1)
                def _prefetch():
                    start(ci + 2, b)
            return _

        lax.fori_loop(0, n_ci // 2, pair, None)
        for b in range(2):
            pltpu.make_async_copy(out_v.at[b], out_ref.at[pl.ds(0, _B_CHUNK)],
                                  so[b]).wait()

    return pl.kernel(
        body,
        out_type=jax.ShapeDtypeStruct((_B_PAD, _H), jnp.float32),
        mesh=_sc_mesh(),
        scratch_types=[
            pltpu.VMEM((_B_PW,), jnp.int32),
            pltpu.VMEM((_B_PW,), jnp.int32),
            pltpu.VMEM((2, _B_CHUNK, _H), jnp.float32),
            pltpu.VMEM((2, _B_CHUNK, _H), jnp.float32),
            pltpu.VMEM((2, _B_CHUNK, _H), jnp.float32),
        ] + [pltpu.SemaphoreType.DMA] * 6,
    )(msg_atom, msg_bond, b2a, b2revb)


def _tc_mm(x, w, add=None, relu=False, out_rows=None, grid_rows=None):
    m, kdim = x.shape
    n = w.shape[1]
    out_rows = out_rows or m
    grid_rows = grid_rows or m
    grid = pl.cdiv(grid_rows, _BM)

    def body(*refs):
        if add is not None:
            x_ref, w_ref, a_ref, o_ref = refs
        else:
            x_ref, w_ref, o_ref = refs
        acc = jnp.dot(x_ref[...], w_ref[...], preferred_element_type=jnp.float32)
        if add is not None:
            acc = acc + a_ref[...]
        if relu:
            acc = jnp.maximum(acc, 0.0)
        o_ref[...] = acc

    in_specs = [
        pl.BlockSpec((_BM, kdim), lambda i: (i, 0)),
        pl.BlockSpec((kdim, n), lambda i: (0, 0)),
    ]
    args = [x, w]
    if add is not None:
        in_specs.append(pl.BlockSpec((_BM, n), lambda i: (i, 0)))
        args.append(add)
    return pl.pallas_call(
        body,
        grid=(grid,),
        in_specs=in_specs,
        out_specs=pl.BlockSpec((_BM, n), lambda i: (i, 0)),
        out_shape=jax.ShapeDtypeStruct((out_rows, n), jnp.float32),
    )(*args)


def _tc_mm3(x1, w1, x2, w2, x3, w3):
    grid = _A_PAD // _BM

    def body(x1_ref, w1_ref, x2_ref, w2_ref, x3_ref, w3_ref, o_ref):
        acc = jnp.dot(x1_ref[...], w1_ref[...], preferred_element_type=jnp.float32)
        acc += jnp.dot(x2_ref[...], w2_ref[...], preferred_element_type=jnp.float32)
        acc += jnp.dot(x3_ref[...], w3_ref[...], preferred_element_type=jnp.float32)
        o_ref[...] = acc

    xspec = pl.BlockSpec((_BM, _H), lambda i: (i, 0))
    wspec = pl.BlockSpec((_H, _H), lambda i: (0, 0))
    return pl.pallas_call(
        body,
        grid=(grid,),
        in_specs=[xspec, wspec, xspec, wspec, xspec, wspec],
        out_specs=xspec,
        out_shape=jax.ShapeDtypeStruct((_A_PAD, _H), jnp.float32),
    )(x1, w1, x2, w2, x3, w3)


def _tc_h0(node_tm):
    tblk = 125
    grid = _APM // tblk

    def body(x_ref, o_ref):
        i = pl.program_id(0)
        bm = jnp.max(x_ref[...], axis=0)

        @pl.when(i == 0)
        def _init():
            o_ref[...] = bm

        @pl.when(i > 0)
        def _acc():
            o_ref[...] = jnp.maximum(o_ref[...], bm)

    return pl.pallas_call(
        body,
        grid=(grid,),
        in_specs=[pl.BlockSpec((tblk, _N_MOL, _H), lambda i: (i, 0, 0))],
        out_specs=pl.BlockSpec((_N_MOL, _H), lambda i: (0, 0)),
        out_shape=jax.ShapeDtypeStruct((_N_MOL, _H), jnp.float32),
    )(node_tm)


def _tc_gi(node_flat, gru_bias, wf_t, bf, wb_t, bb):
    m = node_flat.shape[0]
    grid = pl.cdiv(m, _BM)

    def body(x_ref, gbias_ref, wf_ref, bf_ref, wb_ref, bb_ref, of_ref, ob_ref):
        msg = jnp.maximum(x_ref[...] + gbias_ref[...], 0.0)
        of_ref[...] = jnp.dot(msg, wf_ref[...], preferred_element_type=jnp.float32) + bf_ref[...]
        ob_ref[...] = jnp.dot(msg, wb_ref[...], preferred_element_type=jnp.float32) + bb_ref[...]

    wspec = pl.BlockSpec((_H, 3 * _H), lambda i: (0, 0))
    bspec = pl.BlockSpec((1, 3 * _H), lambda i: (0, 0))
    ospec = pl.BlockSpec((_BM, 3 * _H), lambda i: (i, 0))
    oshape = jax.ShapeDtypeStruct((m, 3 * _H), jnp.float32)
    return pl.pallas_call(
        body,
        grid=(grid,),
        in_specs=[
            pl.BlockSpec((_BM, _H), lambda i: (i, 0)),
            pl.BlockSpec((1, _H), lambda i: (0, 0)),
            wspec, bspec, wspec, bspec,
        ],
        out_specs=[ospec, ospec],
        out_shape=[oshape, oshape],
    )(node_flat, gru_bias, wf_t, bf, wb_t, bb)


def _tc_gru(gi_f, gi_b, h0, whhf_t, whhb_t, bhf, bhb):
    T = _APM
    TSUB = 25
    NBLK = T // TSUB

    def body(gif_ref, gib_ref, h0_ref, wf_ref, wb_ref, bf_ref, bb_ref,
             of_ref, ob_ref, hf, hb):
        t = pl.program_id(0)

        @pl.when(t == 0)
        def _init():
            hf[...] = h0_ref[...]
            hb[...] = h0_ref[...]

        def step(gi, h, w_ref, b_ref):
            gh = jnp.dot(h, w_ref[...], preferred_element_type=jnp.float32) + b_ref[...]
            i_r = gi[:, :_H]
            i_z = gi[:, _H:2 * _H]
            i_n = gi[:, 2 * _H:]
            h_r = gh[:, :_H]
            h_z = gh[:, _H:2 * _H]
            h_n = gh[:, 2 * _H:]
            r = jax.nn.sigmoid(i_r + h_r)
            z = jax.nn.sigmoid(i_z + h_z)
            n = jnp.tanh(i_n + r * h_n)
            return (1.0 - z) * n + z * h

        hfv = hf[...]
        hbv = hb[...]
        for k in range(TSUB):
            hfv = step(gif_ref[k], hfv, wf_ref, bf_ref)
            of_ref[k] = hfv
            kb = TSUB - 1 - k
            hbv = step(gib_ref[kb], hbv, wb_ref, bb_ref)
            ob_ref[kb] = hbv
        hf[...] = hfv
        hb[...] = hbv

    gspec_f = pl.BlockSpec((TSUB, _N_MOL, 3 * _H), lambda t: (t, 0, 0))
    gspec_b = pl.BlockSpec((TSUB, _N_MOL, 3 * _H), lambda t: (NBLK - 1 - t, 0, 0))
    wspec = pl.BlockSpec((_H, 3 * _H), lambda t: (0, 0))
    bspec = pl.BlockSpec((1, 3 * _H), lambda t: (0, 0))
    ospec_f = pl.BlockSpec((TSUB, _N_MOL, _H), lambda t: (t, 0, 0))
    ospec_b = pl.BlockSpec((TSUB, _N_MOL, _H), lambda t: (NBLK - 1 - t, 0, 0))
    oshape = jax.ShapeDtypeStruct((T, _N_MOL, _H), jnp.float32)
    return pl.pallas_call(
        body,
        grid=(NBLK,),
        in_specs=[
            gspec_f, gspec_b,
            pl.BlockSpec((_N_MOL, _H), lambda t: (0, 0)),
            wspec, wspec, bspec, bspec,
        ],
        out_specs=[ospec_f, ospec_b],
        out_shape=[oshape, oshape],
        scratch_shapes=[
            pltpu.VMEM((_N_MOL, _H), jnp.float32),
            pltpu.VMEM((_N_MOL, _H), jnp.float32),
        ],
    )(gi_f, gi_b, h0, whhf_t, whhb_t, bhf, bhb)


def _tc_final(xf, xb, wof, wob, bo):
    tblk = 125
    rblk = tblk * _N_MOL
    grid = _APM // tblk

    def body(xf_ref, xb_ref, wof_ref, wob_ref, bo_ref, o_ref):
        i = pl.program_id(0)
        y = jnp.dot(xf_ref[...], wof_ref[...], preferred_element_type=jnp.float32)
        y += jnp.dot(xb_ref[...], wob_ref[...], preferred_element_type=jnp.float32)
        y = jnp.maximum(y + bo_ref[...], 0.0)
        part = jnp.sum(y.reshape(tblk, _N_MOL, _H), axis=0) * (1.0 / _APM)

        @pl.when(i == 0)
        def _init():
            o_ref[...] = part

        @pl.when(i > 0)
        def _acc():
            o_ref[...] = o_ref[...] + part

    xspec = pl.BlockSpec((rblk, _H), lambda i: (i, 0))
    wspec = pl.BlockSpec((_H, _H), lambda i: (0, 0))
    return pl.pallas_call(
        body,
        grid=(grid,),
        in_specs=[xspec, xspec, wspec, wspec,
                  pl.BlockSpec((1, _H), lambda i: (0, 0))],
        out_specs=pl.BlockSpec((_N_MOL, _H), lambda i: (0, 0)),
        out_shape=jax.ShapeDtypeStruct((_N_MOL, _H), jnp.float32),
    )(xf, xb, wof, wob, bo)


def kernel(f_atoms, f_bonds, a2b, b2a, b2revb, W_i_atom, W_i_bond, W_h_0,
           W_h_1, W_lr, W_o, b_o, gru_bias, gru_w_ih_f, gru_w_hh_f,
           gru_b_ih_f, gru_b_hh_f, gru_w_ih_b, gru_w_hh_b, gru_b_ih_b,
           gru_b_hh_b):
    i32 = jnp.int32
    pad_a = jnp.arange((_A_PAD - _NA) * _MAXNB, dtype=i32) * 37 % _NB
    a2b_flat = jnp.concatenate([a2b.astype(i32).reshape(-1), pad_a])
    pad_ba = jnp.arange(_B_PAD - _NB, dtype=i32) * 2 % _NA
    pad_br = jnp.arange(_B_PAD - _NB, dtype=i32) * 37 % _NB
    b2a_p = jnp.concatenate([b2a.astype(i32), pad_ba])
    b2revb_p = jnp.concatenate([b2revb.astype(i32), pad_br])

    grid_b = pl.cdiv(_NB, _BM) * _BM

    input_atom = _tc_mm(f_atoms, W_i_atom, relu=True,
                        out_rows=_A_PAD, grid_rows=_A_PAD)
    input_bond = _tc_mm(f_bonds, W_i_bond, relu=True,
                        out_rows=_B_PAD, grid_rows=grid_b)

    message_atom = input_atom
    message_bond = input_bond
    for W_h in (W_h_0, W_h_1):
        message_atom = _sc_agg(message_bond, a2b_flat, message_atom)
        tmp = _sc_bond(message_atom, message_bond, b2a_p, b2revb_p)
        message_bond = _tc_mm(tmp, W_h, add=input_bond, relu=True,
                              out_rows=_B_PAD, grid_rows=grid_b)

    agg = _sc_agg(message_bond, a2b_flat, None)
    node = _tc_mm3(agg, W_lr[:_H], message_atom, W_lr[_H:2 * _H],
                   input_atom, W_lr[2 * _H:])

    node_tm = node[1:_NA].reshape(_N_MOL, _APM, _H).transpose(1, 0, 2)
    h0 = _tc_h0(node_tm)
    gi_f, gi_b = _tc_gi(
        node_tm.reshape(_N_MOL * _APM, _H), gru_bias.reshape(1, _H),
        gru_w_ih_f.T, gru_b_ih_f.reshape(1, 3 * _H),
        gru_w_ih_b.T, gru_b_ih_b.reshape(1, 3 * _H))
    out_f, out_b = _tc_gru(
        gi_f.reshape(_APM, _N_MOL, 3 * _H), gi_b.reshape(_APM, _N_MOL, 3 * _H),
        h0, gru_w_hh_f.T, gru_w_hh_b.T,
        gru_b_hh_f.reshape(1, 3 * _H), gru_b_hh_b.reshape(1, 3 * _H))

    return _tc_final(out_f.reshape(-1, _H), out_b.reshape(-1, _H),
                     W_o[:_H], W_o[_H:], b_o.reshape(1, _H))


# speedup vs baseline: 4.9653x; 1.1668x over previous
import functools

import jax
import jax.numpy as jnp
from jax import lax
from jax.experimental import pallas as pl
from jax.experimental.pallas import tpu as pltpu
from jax.experimental.pallas import tpu_sc as plsc

_N_MOL = 16
_APM = 625
_NA = 1 + _N_MOL * _APM
_MAXNB = 16
_NB = 1 + _N_MOL * _APM * _MAXNB
_H = 256

_NW = 32
_LANES = 16

_A_CHUNK = 8
_A_CHUNKS_PW = 40
_A_PW = _A_CHUNK * _A_CHUNKS_PW
_A_PAD = _NW * _A_PW

_B_CHUNK = 64
_B_CHUNKS_PW = 80
_B_PW = _B_CHUNK * _B_CHUNKS_PW
_B_PAD = _NW * _B_PW

_BM = 1024


def _sc_mesh():
    return plsc.VectorSubcoreMesh(core_axis_name="c", subcore_axis_name="s")


def _sc_agg(msg_bond, a2b_flat, base):
    add_base = base is not None
    n_idx = _A_CHUNK * _MAXNB
    n_ci = _A_CHUNKS_PW

    def body(*refs):
        if add_base:
            (msg_ref, idx_ref, base_ref, out_ref, idx_all, rows_v, base_v,
             out_v, sg0, sg1, sb0, sb1, so0, so1) = refs
        else:
            (msg_ref, idx_ref, out_ref, idx_all, rows_v,
             out_v, sg0, sg1, sb0, sb1, so0, so1) = refs
        sg = (sg0, sg1)
        sb = (sb0, sb1)
        so = (so0, so1)
        nc = lax.axis_size("c")
        wid = lax.axis_index("s") * nc + lax.axis_index("c")
        wbase = wid * _A_PW

        pltpu.sync_copy(idx_ref.at[pl.ds(wbase * _MAXNB, _A_PW * _MAXNB)],
                        idx_all)

        def start(ci, b):
            pltpu.async_copy(
                msg_ref.at[idx_all.at[pl.ds(ci * n_idx, n_idx)]],
                rows_v.at[b], sg[b])
            if add_base:
                pltpu.async_copy(
                    base_ref.at[pl.ds(wbase + ci * _A_CHUNK, _A_CHUNK)],
                    base_v.at[b], sb[b])

        def wait_in(b):
            pltpu.make_async_copy(msg_ref.at[pl.ds(0, n_idx)],
                                  rows_v.at[b], sg[b]).wait()
            if add_base:
                pltpu.make_async_copy(base_ref.at[pl.ds(0, _A_CHUNK)],
                                      base_v.at[b], sb[b]).wait()

        def compute(ci, b):
            def per_atom(a, _):
                r0 = a * _MAXNB
                for c in range(_H // _LANES):
                    sl = pl.ds(c * _LANES, _LANES)
                    v = [rows_v[b, r0 + j, sl] for j in range(_MAXNB)]
                    m = v[0]
                    for j in range(1, _MAXNB):
                        m = jnp.maximum(m, v[j])
                    n = _MAXNB
                    while n > 1:
                        h = n // 2
                        v = [v[j] + v[j + h] for j in range(h)]
                        n = h
                    res = v[0] * m
                    if add_base:
                        res = base_v[b, a, sl] + res
                    out_v[b, a, sl] = res
                return _

            lax.fori_loop(0, _A_CHUNK, per_atom, None)

        for b in range(2):
            start(b, b)

        def pair(p, _):
            for b in range(2):
                ci = p * 2 + b
                wait_in(b)

                @pl.when(p > 0)
                def _drain_out():
                    pltpu.make_async_copy(
                        out_v.at[b], out_ref.at[pl.ds(0, _A_CHUNK)],
                        so[b]).wait()

                compute(ci, b)
                pltpu.async_copy(out_v.at[b],
                                 out_ref.at[pl.ds(wbase + ci * _A_CHUNK,
                                                  _A_CHUNK)], so[b])

                @pl.when(p < n_ci // 2 - 1)
                def _prefetch():
                    start(ci + 2, b)
            return _

        lax.fori_loop(0, n_ci // 2, pair, None)
        for b in range(2):
            pltpu.make_async_copy(out_v.at[b], out_ref.at[pl.ds(0, _A_CHUNK)],
                                  so[b]).wait()

    scratch = [
        pltpu.VMEM((_A_PW * _MAXNB,), jnp.int32),
        pltpu.VMEM((2, n_idx, _H), jnp.float32),
    ]
    if add_base:
        scratch.append(pltpu.VMEM((2, _A_CHUNK, _H), jnp.float32))
    scratch += [
        pltpu.VMEM((2, _A_CHUNK, _H), jnp.float32),
    ] + [pltpu.SemaphoreType.DMA] * 6
    k = pl.kernel(
        body,
        out_type=jax.ShapeDtypeStruct((_A_PAD, _H), jnp.float32),
        mesh=_sc_mesh(),
        scratch_types=scratch,
    )
    if add_base:
        return k(msg_bond, a2b_flat, base)
    return k(msg_bond, a2b_flat)


def _sc_bond(msg_atom, msg_bond, b2a, b2revb):
    n_ci = _B_CHUNKS_PW

    def body(atom_ref, bond_ref, b2a_ref, b2revb_ref, out_ref,
             idxa_all, idxb_all, rows_a, rows_b, out_v,
             sa0, sa1, sb0, sb1, so0, so1):
        sa = (sa0, sa1)
        sb = (sb0, sb1)
        so = (so0, so1)
        nc = lax.axis_size("c")
        wid = lax.axis_index("s") * nc + lax.axis_index("c")
        wbase = wid * _B_PW

        pltpu.sync_copy(b2a_ref.at[pl.ds(wbase, _B_PW)], idxa_all)
        pltpu.sync_copy(b2revb_ref.at[pl.ds(wbase, _B_PW)], idxb_all)

        def start(ci, b):
            pltpu.async_copy(
                atom_ref.at[idxa_all.at[pl.ds(ci * _B_CHUNK, _B_CHUNK)]],
                rows_a.at[b], sa[b])
            pltpu.async_copy(
                bond_ref.at[idxb_all.at[pl.ds(ci * _B_CHUNK, _B_CHUNK)]],
                rows_b.at[b], sb[b])

        def wait_in(b):
            pltpu.make_async_copy(atom_ref.at[pl.ds(0, _B_CHUNK)],
                                  rows_a.at[b], sa[b]).wait()
            pltpu.make_async_copy(bond_ref.at[pl.ds(0, _B_CHUNK)],
                                  rows_b.at[b], sb[b]).wait()

        def compute(b):
            def per_row(r, _):
                for c in range(_H // _LANES):
                    sl = pl.ds(c * _LANES, _LANES)
                    out_v[b, r, sl] = rows_a[b, r, sl] - rows_b[b, r, sl]
                return _

            lax.fori_loop(0, _B_CHUNK, per_row, None)

        for b in range(2):
            start(b, b)

        def pair(p, _):
            for b in range(2):
                ci = p * 2 + b
                wait_in(b)

                @pl.when(p > 0)
                def _drain_out():
                    pltpu.make_async_copy(
                        out_v.at[b], out_ref.at[pl.ds(0, _B_CHUNK)],
                        so[b]).wait()

                compute(b)
                pltpu.async_copy(out_v.at[b],
                                 out_ref.at[pl.ds(wbase + ci * _B_CHUNK,
                                                  _B_CHUNK)], so[b])

                @pl.when(p < n_ci // 2 - 1)
                def _prefetch():
                    start(ci + 2, b)
            return _

        lax.fori_loop(0, n_ci // 2, pair, None)
        for b in range(2):
            pltpu.make_async_copy(out_v.at[b], out_ref.at[pl.ds(0, _B_CHUNK)],
                                  so[b]).wait()

    return pl.kernel(
        body,
        out_type=jax.ShapeDtypeStruct((_B_PAD, _H), jnp.float32),
        mesh=_sc_mesh(),
        scratch_types=[
            pltpu.VMEM((_B_PW,), jnp.int32),
            pltpu.VMEM((_B_PW,), jnp.int32),
            pltpu.VMEM((2, _B_CHUNK, _H), jnp.float32),
            pltpu.VMEM((2, _B_CHUNK, _H), jnp.float32),
            pltpu.VMEM((2, _B_CHUNK, _H), jnp.float32),
        ] + [pltpu.SemaphoreType.DMA] * 6,
    )(msg_atom, msg_bond, b2a, b2revb)


def _tc_mm(x, w, add=None, relu=False, out_rows=None, grid_rows=None):
    m, kdim = x.shape
    n = w.shape[1]
    out_rows = out_rows or m
    grid_rows = grid_rows or m
    grid = pl.cdiv(grid_rows, _BM)

    def body(*refs):
        if add is not None:
            x_ref, w_ref, a_ref, o_ref = refs
        else:
            x_ref, w_ref, o_ref = refs
        acc = jnp.dot(x_ref[...], w_ref[...], preferred_element_type=jnp.float32)
        if add is not None:
            acc = acc + a_ref[...]
        if relu:
            acc = jnp.maximum(acc, 0.0)
        o_ref[...] = acc

    in_specs = [
        pl.BlockSpec((_BM, kdim), lambda i: (i, 0)),
        pl.BlockSpec((kdim, n), lambda i: (0, 0)),
    ]
    args = [x, w]
    if add is not None:
        in_specs.append(pl.BlockSpec((_BM, n), lambda i: (i, 0)))
        args.append(add)
    return pl.pallas_call(
        body,
        grid=(grid,),
        in_specs=in_specs,
        out_specs=pl.BlockSpec((_BM, n), lambda i: (i, 0)),
        out_shape=jax.ShapeDtypeStruct((out_rows, n), jnp.float32),
    )(*args)


def _tc_mm3(x1, w1, x2, w2, x3, w3):
    grid = _A_PAD // _BM

    def body(x1_ref, w1_ref, x2_ref, w2_ref, x3_ref, w3_ref, o_ref):
        acc = jnp.dot(x1_ref[...], w1_ref[...], preferred_element_type=jnp.float32)
        acc += jnp.dot(x2_ref[...], w2_ref[...], preferred_element_type=jnp.float32)
        acc += jnp.dot(x3_ref[...], w3_ref[...], preferred_element_type=jnp.float32)
        o_ref[...] = acc

    xspec = pl.BlockSpec((_BM, _H), lambda i: (i, 0))
    wspec = pl.BlockSpec((_H, _H), lambda i: (0, 0))
    return pl.pallas_call(
        body,
        grid=(grid,),
        in_specs=[xspec, wspec, xspec, wspec, xspec, wspec],
        out_specs=xspec,
        out_shape=jax.ShapeDtypeStruct((_A_PAD, _H), jnp.float32),
    )(x1, w1, x2, w2, x3, w3)


def _tc_h0(node_tm):
    tblk = 125
    grid = _APM // tblk

    def body(x_ref, o_ref):
        i = pl.program_id(0)
        bm = jnp.max(x_ref[...], axis=0)

        @pl.when(i == 0)
        def _init():
            o_ref[...] = bm

        @pl.when(i > 0)
        def _acc():
            o_ref[...] = jnp.maximum(o_ref[...], bm)

    return pl.pallas_call(
        body,
        grid=(grid,),
        in_specs=[pl.BlockSpec((tblk, _N_MOL, _H), lambda i: (i, 0, 0))],
        out_specs=pl.BlockSpec((_N_MOL, _H), lambda i: (0, 0)),
        out_shape=jax.ShapeDtypeStruct((_N_MOL, _H), jnp.float32),
    )(node_tm)


def _tc_gi(node_flat, gru_bias, wf_t, bf, wb_t, bb):
    m = node_flat.shape[0]
    grid = pl.cdiv(m, _BM)

    def body(x_ref, gbias_ref, wf_ref, bf_ref, wb_ref, bb_ref, of_ref, ob_ref):
        msg = jnp.maximum(x_ref[...] + gbias_ref[...], 0.0)
        of_ref[...] = jnp.dot(msg, wf_ref[...], preferred_element_type=jnp.float32) + bf_ref[...]
        ob_ref[...] = jnp.dot(msg, wb_ref[...], preferred_element_type=jnp.float32) + bb_ref[...]

    wspec = pl.BlockSpec((_H, 3 * _H), lambda i: (0, 0))
    bspec = pl.BlockSpec((1, 3 * _H), lambda i: (0, 0))
    ospec = pl.BlockSpec((_BM, 3 * _H), lambda i: (i, 0))
    oshape = jax.ShapeDtypeStruct((m, 3 * _H), jnp.float32)
    return pl.pallas_call(
        body,
        grid=(grid,),
        in_specs=[
            pl.BlockSpec((_BM, _H), lambda i: (i, 0)),
            pl.BlockSpec((1, _H), lambda i: (0, 0)),
            wspec, bspec, wspec, bspec,
        ],
        out_specs=[ospec, ospec],
        out_shape=[oshape, oshape],
    )(node_flat, gru_bias, wf_t, bf, wb_t, bb)


def _tc_gru(gi_f, gi_b, h0, whhf_t, whhb_t, bhf, bhb):
    T = _APM
    TSUB = 25
    NBLK = T // TSUB

    def body(gif_ref, gib_ref, h0_ref, wf_ref, wb_ref, bf_ref, bb_ref,
             of_ref, ob_ref, hf, hb):
        t = pl.program_id(0)

        @pl.when(t == 0)
        def _init():
            hf[...] = h0_ref[...]
            hb[...] = h0_ref[...]

        def step(gi, h, w_ref, b_ref):
            gh = jnp.dot(h, w_ref[...], preferred_element_type=jnp.float32) + b_ref[...]
            i_r = gi[:, :_H]
            i_z = gi[:, _H:2 * _H]
            i_n = gi[:, 2 * _H:]
            h_r = gh[:, :_H]
            h_z = gh[:, _H:2 * _H]
            h_n = gh[:, 2 * _H:]
            r = jax.nn.sigmoid(i_r + h_r)
            z = jax.nn.sigmoid(i_z + h_z)
            n = jnp.tanh(i_n + r * h_n)
            return (1.0 - z) * n + z * h

        hfv = hf[...]
        hbv = hb[...]
        for k in range(TSUB):
            hfv = step(gif_ref[k], hfv, wf_ref, bf_ref)
            of_ref[k] = hfv
            kb = TSUB - 1 - k
            hbv = step(gib_ref[kb], hbv, wb_ref, bb_ref)
            ob_ref[kb] = hbv
        hf[...] = hfv
        hb[...] = hbv

    gspec_f = pl.BlockSpec((TSUB, _N_MOL, 3 * _H), lambda t: (t, 0, 0))
    gspec_b = pl.BlockSpec((TSUB, _N_MOL, 3 * _H), lambda t: (NBLK - 1 - t, 0, 0))
    wspec = pl.BlockSpec((_H, 3 * _H), lambda t: (0, 0))
    bspec = pl.BlockSpec((1, 3 * _H), lambda t: (0, 0))
    ospec_f = pl.BlockSpec((TSUB, _N_MOL, _H), lambda t: (t, 0, 0))
    ospec_b = pl.BlockSpec((TSUB, _N_MOL, _H), lambda t: (NBLK - 1 - t, 0, 0))
    oshape = jax.ShapeDtypeStruct((T, _N_MOL, _H), jnp.float32)
    return pl.pallas_call(
        body,
        grid=(NBLK,),
        in_specs=[
            gspec_f, gspec_b,
            pl.BlockSpec((_N_MOL, _H), lambda t: (0, 0)),
            wspec, wspec, bspec, bspec,
        ],
        out_specs=[ospec_f, ospec_b],
        out_shape=[oshape, oshape],
        scratch_shapes=[
            pltpu.VMEM((_N_MOL, _H), jnp.float32),
            pltpu.VMEM((_N_MOL, _H), jnp.float32),
        ],
    )(gi_f, gi_b, h0, whhf_t, whhb_t, bhf, bhb)


def _tc_final(xf, xb, wof, wob, bo):
    tblk = 125
    rblk = tblk * _N_MOL
    grid = _APM // tblk

    def body(xf_ref, xb_ref, wof_ref, wob_ref, bo_ref, o_ref):
        i = pl.program_id(0)
        y = jnp.dot(xf_ref[...], wof_ref[...], preferred_element_type=jnp.float32)
        y += jnp.dot(xb_ref[...], wob_ref[...], preferred_element_type=jnp.float32)
        y = jnp.maximum(y + bo_ref[...], 0.0)
        part = jnp.sum(y.reshape(tblk, _N_MOL, _H), axis=0) * (1.0 / _APM)

        @pl.when(i == 0)
        def _init():
            o_ref[...] = part

        @pl.when(i > 0)
        def _acc():
            o_ref[...] = o_ref[...] + part

    xspec = pl.BlockSpec((rblk, _H), lambda i: (i, 0))
    wspec = pl.BlockSpec((_H, _H), lambda i: (0, 0))
    return pl.pallas_call(
        body,
        grid=(grid,),
        in_specs=[xspec, xspec, wspec, wspec,
                  pl.BlockSpec((1, _H), lambda i: (0, 0))],
        out_specs=pl.BlockSpec((_N_MOL, _H), lambda i: (0, 0)),
        out_shape=jax.ShapeDtypeStruct((_N_MOL, _H), jnp.float32),
    )(xf, xb, wof, wob, bo)


def kernel(f_atoms, f_bonds, a2b, b2a, b2revb, W_i_atom, W_i_bond, W_h_0,
           W_h_1, W_lr, W_o, b_o, gru_bias, gru_w_ih_f, gru_w_hh_f,
           gru_b_ih_f, gru_b_hh_f, gru_w_ih_b, gru_w_hh_b, gru_b_ih_b,
           gru_b_hh_b):
    i32 = jnp.int32
    pad_a = jnp.arange((_A_PAD - _NA) * _MAXNB, dtype=i32) * 37 % _NB
    a2b_flat = jnp.concatenate([a2b.astype(i32).reshape(-1), pad_a])
    pad_ba = jnp.arange(_B_PAD - _NB, dtype=i32) * 2 % _NA
    pad_br = jnp.arange(_B_PAD - _NB, dtype=i32) * 37 % _NB
    b2a_p = jnp.concatenate([b2a.astype(i32), pad_ba])
    b2revb_p = jnp.concatenate([b2revb.astype(i32), pad_br])

    grid_b = pl.cdiv(_NB, _BM) * _BM

    input_atom = _tc_mm(f_atoms, W_i_atom, relu=True,
                        out_rows=_A_PAD, grid_rows=_A_PAD)
    input_bond = _tc_mm(f_bonds, W_i_bond, relu=True,
                        out_rows=_B_PAD, grid_rows=grid_b)

    message_atom = input_atom
    message_bond = input_bond
    for W_h in (W_h_0, W_h_1):
        message_atom = _sc_agg(message_bond, a2b_flat, message_atom)
        tmp = _sc_bond(message_atom, message_bond, b2a_p, b2revb_p)
        message_bond = _tc_mm(tmp, W_h, add=input_bond, relu=True,
                              out_rows=_B_PAD, grid_rows=grid_b)

    agg = _sc_agg(message_bond, a2b_flat, None)
    node = _tc_mm3(agg, W_lr[:_H], message_atom, W_lr[_H:2 * _H],
                   input_atom, W_lr[2 * _H:])

    node_tm = node[1:_NA].reshape(_N_MOL, _APM, _H).transpose(1, 0, 2)
    h0 = _tc_h0(node_tm)
    gi_f, gi_b = _tc_gi(
        node_tm.reshape(_N_MOL * _APM, _H), gru_bias.reshape(1, _H),
        gru_w_ih_f.T, gru_b_ih_f.reshape(1, 3 * _H),
        gru_w_ih_b.T, gru_b_ih_b.reshape(1, 3 * _H))
    out_f, out_b = _tc_gru(
        gi_f.reshape(_APM, _N_MOL, 3 * _H), gi_b.reshape(_APM, _N_MOL, 3 * _H),
        h0, gru_w_hh_f.T, gru_w_hh_b.T,
        gru_b_hh_f.reshape(1, 3 * _H), gru_b_hh_b.reshape(1, 3 * _H))

    return _tc_final(out_f.reshape(-1, _H), out_b.reshape(-1, _H),
                     W_o[:_H], W_o[_H:], b_o.reshape(1, _H))


# speedup vs baseline: 5.3770x; 1.0829x over previous
import functools

import jax
import jax.numpy as jnp
from jax import lax
from jax.experimental import pallas as pl
from jax.experimental.pallas import tpu as pltpu
from jax.experimental.pallas import tpu_sc as plsc

_N_MOL = 16
_APM = 625
_NA = 1 + _N_MOL * _APM
_MAXNB = 16
_NB = 1 + _N_MOL * _APM * _MAXNB
_H = 256

_NW = 32
_LANES = 16

_A_CHUNK = 8
_A_CHUNKS_PW = 40
_A_PW = _A_CHUNK * _A_CHUNKS_PW
_A_PAD = _NW * _A_PW

_B_CHUNK = 64
_B_CHUNKS_PW = 80
_B_PW = _B_CHUNK * _B_CHUNKS_PW
_B_PAD = _NW * _B_PW

_BM = 2048


def _sc_mesh():
    return plsc.VectorSubcoreMesh(core_axis_name="c", subcore_axis_name="s")


def _sc_agg(msg_bond, a2b_flat, base):
    add_base = base is not None
    n_idx = _A_CHUNK * _MAXNB
    n_ci = _A_CHUNKS_PW

    def body(*refs):
        if add_base:
            (msg_ref, idx_ref, base_ref, out_ref, idx_all, rows_v, base_v,
             out_v, sg0, sg1, sb0, sb1, so0, so1) = refs
        else:
            (msg_ref, idx_ref, out_ref, idx_all, rows_v,
             out_v, sg0, sg1, sb0, sb1, so0, so1) = refs
        sg = (sg0, sg1)
        sb = (sb0, sb1)
        so = (so0, so1)
        nc = lax.axis_size("c")
        wid = lax.axis_index("s") * nc + lax.axis_index("c")
        wbase = wid * _A_PW

        pltpu.sync_copy(idx_ref.at[pl.ds(wbase * _MAXNB, _A_PW * _MAXNB)],
                        idx_all)

        def start(ci, b):
            pltpu.async_copy(
                msg_ref.at[idx_all.at[pl.ds(ci * n_idx, n_idx)]],
                rows_v.at[b], sg[b])
            if add_base:
                pltpu.async_copy(
                    base_ref.at[pl.ds(wbase + ci * _A_CHUNK, _A_CHUNK)],
                    base_v.at[b], sb[b])

        def wait_in(b):
            pltpu.make_async_copy(msg_ref.at[pl.ds(0, n_idx)],
                                  rows_v.at[b], sg[b]).wait()
            if add_base:
                pltpu.make_async_copy(base_ref.at[pl.ds(0, _A_CHUNK)],
                                      base_v.at[b], sb[b]).wait()

        def compute(ci, b):
            def per_atom(a, _):
                r0 = a * _MAXNB
                for c in range(_H // _LANES):
                    sl = pl.ds(c * _LANES, _LANES)
                    v = [rows_v[b, r0 + j, sl] for j in range(_MAXNB)]
                    m = v[0]
                    for j in range(1, _MAXNB):
                        m = jnp.maximum(m, v[j])
                    n = _MAXNB
                    while n > 1:
                        h = n // 2
                        v = [v[j] + v[j + h] for j in range(h)]
                        n = h
                    res = v[0] * m
                    if add_base:
                        res = base_v[b, a, sl] + res
                    out_v[b, a, sl] = res
                return _

            lax.fori_loop(0, _A_CHUNK, per_atom, None)

        for b in range(2):
            start(b, b)

        def pair(p, _):
            for b in range(2):
                ci = p * 2 + b
                wait_in(b)

                @pl.when(p > 0)
                def _drain_out():
                    pltpu.make_async_copy(
                        out_v.at[b], out_ref.at[pl.ds(0, _A_CHUNK)],
                        so[b]).wait()

                compute(ci, b)
                pltpu.async_copy(out_v.at[b],
                                 out_ref.at[pl.ds(wbase + ci * _A_CHUNK,
                                                  _A_CHUNK)], so[b])

                @pl.when(p < n_ci // 2 - 1)
                def _prefetch():
                    start(ci + 2, b)
            return _

        lax.fori_loop(0, n_ci // 2, pair, None)
        for b in range(2):
            pltpu.make_async_copy(out_v.at[b], out_ref.at[pl.ds(0, _A_CHUNK)],
                                  so[b]).wait()

    scratch = [
        pltpu.VMEM((_A_PW * _MAXNB,), jnp.int32),
        pltpu.VMEM((2, n_idx, _H), jnp.float32),
    ]
    if add_base:
        scratch.append(pltpu.VMEM((2, _A_CHUNK, _H), jnp.float32))
    scratch += [
        pltpu.VMEM((2, _A_CHUNK, _H), jnp.float32),
    ] + [pltpu.SemaphoreType.DMA] * 6
    k = pl.kernel(
        body,
        out_type=jax.ShapeDtypeStruct((_A_PAD, _H), jnp.float32),
        mesh=_sc_mesh(),
        scratch_types=scratch,
    )
    if add_base:
        return k(msg_bond, a2b_flat, base)
    return k(msg_bond, a2b_flat)


def _sc_bond(msg_atom, msg_bond, b2a, b2revb):
    n_ci = _B_CHUNKS_PW

    def body(atom_ref, bond_ref, b2a_ref, b2revb_ref, out_ref,
             idxa_all, idxb_all, rows_a, rows_b, out_v,
             sa0, sa1, sb0, sb1, so0, so1):
        sa = (sa0, sa1)
        sb = (sb0, sb1)
        so = (so0, so1)
        nc = lax.axis_size("c")
        wid = lax.axis_index("s") * nc + lax.axis_index("c")
        wbase = wid * _B_PW

        pltpu.sync_copy(b2a_ref.at[pl.ds(wbase, _B_PW)], idxa_all)
        pltpu.sync_copy(b2revb_ref.at[pl.ds(wbase, _B_PW)], idxb_all)

        def start(ci, b):
            pltpu.async_copy(
                atom_ref.at[idxa_all.at[pl.ds(ci * _B_CHUNK, _B_CHUNK)]],
                rows_a.at[b], sa[b])
            pltpu.async_copy(
                bond_ref.at[idxb_all.at[pl.ds(ci * _B_CHUNK, _B_CHUNK)]],
                rows_b.at[b], sb[b])

        def wait_in(b):
            pltpu.make_async_copy(atom_ref.at[pl.ds(0, _B_CHUNK)],
                                  rows_a.at[b], sa[b]).wait()
            pltpu.make_async_copy(bond_ref.at[pl.ds(0, _B_CHUNK)],
                                  rows_b.at[b], sb[b]).wait()

        def compute(b):
            def per_row(r, _):
                for c in range(_H // _LANES):
                    sl = pl.ds(c * _LANES, _LANES)
                    out_v[b, r, sl] = rows_a[b, r, sl] - rows_b[b, r, sl]
                return _

            lax.fori_loop(0, _B_CHUNK, per_row, None)

        for b in range(2):
            start(b, b)

        def pair(p, _):
            for b in range(2):
                ci = p * 2 + b
                wait_in(b)

                @pl.when(p > 0)
                def _drain_out():
                    pltpu.make_async_copy(
                        out_v.at[b], out_ref.at[pl.ds(0, _B_CHUNK)],
                        so[b]).wait()

                compute(b)
                pltpu.async_copy(out_v.at[b],
                                 out_ref.at[pl.ds(wbase + ci * _B_CHUNK,
                                                  _B_CHUNK)], so[b])

                @pl.when(p < n_ci // 2 - 1)
                def _prefetch():
                    start(ci + 2, b)
            return _

        lax.fori_loop(0, n_ci // 2, pair, None)
        for b in range(2):
            pltpu.make_async_copy(out_v.at[b], out_ref.at[pl.ds(0, _B_CHUNK)],
                                  so[b]).wait()

    return pl.kernel(
        body,
        out_type=jax.ShapeDtypeStruct((_B_PAD, _H), jnp.float32),
        mesh=_sc_mesh(),
        scratch_types=[
            pltpu.VMEM((_B_PW,), jnp.int32),
            pltpu.VMEM((_B_PW,), jnp.int32),
            pltpu.VMEM((2, _B_CHUNK, _H), jnp.float32),
            pltpu.VMEM((2, _B_CHUNK, _H), jnp.float32),
            pltpu.VMEM((2, _B_CHUNK, _H), jnp.float32),
        ] + [pltpu.SemaphoreType.DMA] * 6,
    )(msg_atom, msg_bond, b2a, b2revb)


def _tc_mm(x, w, add=None, relu=False, out_rows=None, grid_rows=None):
    m, kdim = x.shape
    n = w.shape[1]
    out_rows = out_rows or m
    grid_rows = grid_rows or m
    grid = pl.cdiv(grid_rows, _BM)

    def body(*refs):
        if add is not None:
            x_ref, w_ref, a_ref, o_ref = refs
        else:
            x_ref, w_ref, o_ref = refs
        acc = jnp.dot(x_ref[...], w_ref[...], preferred_element_type=jnp.float32)
        if add is not None:
            acc = acc + a_ref[...]
        if relu:
            acc = jnp.maximum(acc, 0.0)
        o_ref[...] = acc

    in_specs = [
        pl.BlockSpec((_BM, kdim), lambda i: (i, 0)),
        pl.BlockSpec((kdim, n), lambda i: (0, 0)),
    ]
    args = [x, w]
    if add is not None:
        in_specs.append(pl.BlockSpec((_BM, n), lambda i: (i, 0)))
        args.append(add)
    return pl.pallas_call(
        body,
        grid=(grid,),
        in_specs=in_specs,
        out_specs=pl.BlockSpec((_BM, n), lambda i: (i, 0)),
        out_shape=jax.ShapeDtypeStruct((out_rows, n), jnp.float32),
    )(*args)


def _tc_mm3(x1, w1, x2, w2, x3, w3):
    grid = _A_PAD // _BM

    def body(x1_ref, w1_ref, x2_ref, w2_ref, x3_ref, w3_ref, o_ref):
        acc = jnp.dot(x1_ref[...], w1_ref[...], preferred_element_type=jnp.float32)
        acc += jnp.dot(x2_ref[...], w2_ref[...], preferred_element_type=jnp.float32)
        acc += jnp.dot(x3_ref[...], w3_ref[...], preferred_element_type=jnp.float32)
        o_ref[...] = acc

    xspec = pl.BlockSpec((_BM, _H), lambda i: (i, 0))
    wspec = pl.BlockSpec((_H, _H), lambda i: (0, 0))
    return pl.pallas_call(
        body,
        grid=(grid,),
        in_specs=[xspec, wspec, xspec, wspec, xspec, wspec],
        out_specs=xspec,
        out_shape=jax.ShapeDtypeStruct((_A_PAD, _H), jnp.float32),
    )(x1, w1, x2, w2, x3, w3)


def _tc_h0(node_tm):
    tblk = 125
    grid = _APM // tblk

    def body(x_ref, o_ref):
        i = pl.program_id(0)
        bm = jnp.max(x_ref[...], axis=0)

        @pl.when(i == 0)
        def _init():
            o_ref[...] = bm

        @pl.when(i > 0)
        def _acc():
            o_ref[...] = jnp.maximum(o_ref[...], bm)

    return pl.pallas_call(
        body,
        grid=(grid,),
        in_specs=[pl.BlockSpec((tblk, _N_MOL, _H), lambda i: (i, 0, 0))],
        out_specs=pl.BlockSpec((_N_MOL, _H), lambda i: (0, 0)),
        out_shape=jax.ShapeDtypeStruct((_N_MOL, _H), jnp.float32),
    )(node_tm)


def _tc_gi(node_flat, gru_bias, wf_t, bf, wb_t, bb):
    m = node_flat.shape[0]
    grid = pl.cdiv(m, _BM)

    def body(x_ref, gbias_ref, wf_ref, bf_ref, wb_ref, bb_ref, of_ref, ob_ref):
        msg = jnp.maximum(x_ref[...] + gbias_ref[...], 0.0)
        of_ref[...] = jnp.dot(msg, wf_ref[...], preferred_element_type=jnp.float32) + bf_ref[...]
        ob_ref[...] = jnp.dot(msg, wb_ref[...], preferred_element_type=jnp.float32) + bb_ref[...]

    wspec = pl.BlockSpec((_H, 3 * _H), lambda i: (0, 0))
    bspec = pl.BlockSpec((1, 3 * _H), lambda i: (0, 0))
    ospec = pl.BlockSpec((_BM, 3 * _H), lambda i: (i, 0))
    oshape = jax.ShapeDtypeStruct((m, 3 * _H), jnp.float32)
    return pl.pallas_call(
        body,
        grid=(grid,),
        in_specs=[
            pl.BlockSpec((_BM, _H), lambda i: (i, 0)),
            pl.BlockSpec((1, _H), lambda i: (0, 0)),
            wspec, bspec, wspec, bspec,
        ],
        out_specs=[ospec, ospec],
        out_shape=[oshape, oshape],
    )(node_flat, gru_bias, wf_t, bf, wb_t, bb)


def _tc_gru(gi_f, gi_b, h0, whhf_t, whhb_t, bhf, bhb):
    T = _APM
    TSUB = 25
    NBLK = T // TSUB

    def body(gif_ref, gib_ref, h0_ref, wf_ref, wb_ref, bf_ref, bb_ref,
             of_ref, ob_ref, hf, hb):
        t = pl.program_id(0)

        @pl.when(t == 0)
        def _init():
            hf[...] = h0_ref[...]
            hb[...] = h0_ref[...]

        def step(gi, h, w_ref, b_ref):
            gh = jnp.dot(h, w_ref[...], preferred_element_type=jnp.float32) + b_ref[...]
            i_r = gi[:, :_H]
            i_z = gi[:, _H:2 * _H]
            i_n = gi[:, 2 * _H:]
            h_r = gh[:, :_H]
            h_z = gh[:, _H:2 * _H]
            h_n = gh[:, 2 * _H:]
            r = jax.nn.sigmoid(i_r + h_r)
            z = jax.nn.sigmoid(i_z + h_z)
            n = jnp.tanh(i_n + r * h_n)
            return (1.0 - z) * n + z * h

        hfv = hf[...]
        hbv = hb[...]
        for k in range(TSUB):
            hfv = step(gif_ref[k], hfv, wf_ref, bf_ref)
            of_ref[k] = hfv
            kb = TSUB - 1 - k
            hbv = step(gib_ref[kb], hbv, wb_ref, bb_ref)
            ob_ref[kb] = hbv
        hf[...] = hfv
        hb[...] = hbv

    gspec_f = pl.BlockSpec((TSUB, _N_MOL, 3 * _H), lambda t: (t, 0, 0))
    gspec_b = pl.BlockSpec((TSUB, _N_MOL, 3 * _H), lambda t: (NBLK - 1 - t, 0, 0))
    wspec = pl.BlockSpec((_H, 3 * _H), lambda t: (0, 0))
    bspec = pl.BlockSpec((1, 3 * _H), lambda t: (0, 0))
    ospec_f = pl.BlockSpec((TSUB, _N_MOL, _H), lambda t: (t, 0, 0))
    ospec_b = pl.BlockSpec((TSUB, _N_MOL, _H), lambda t: (NBLK - 1 - t, 0, 0))
    oshape = jax.ShapeDtypeStruct((T, _N_MOL, _H), jnp.float32)
    return pl.pallas_call(
        body,
        grid=(NBLK,),
        in_specs=[
            gspec_f, gspec_b,
            pl.BlockSpec((_N_MOL, _H), lambda t: (0, 0)),
            wspec, wspec, bspec, bspec,
        ],
        out_specs=[ospec_f, ospec_b],
        out_shape=[oshape, oshape],
        scratch_shapes=[
            pltpu.VMEM((_N_MOL, _H), jnp.float32),
            pltpu.VMEM((_N_MOL, _H), jnp.float32),
        ],
    )(gi_f, gi_b, h0, whhf_t, whhb_t, bhf, bhb)


def _tc_final(xf, xb, wof, wob, bo):
    tblk = 125
    rblk = tblk * _N_MOL
    grid = _APM // tblk

    def body(xf_ref, xb_ref, wof_ref, wob_ref, bo_ref, o_ref):
        i = pl.program_id(0)
        y = jnp.dot(xf_ref[...], wof_ref[...], preferred_element_type=jnp.float32)
        y += jnp.dot(xb_ref[...], wob_ref[...], preferred_element_type=jnp.float32)
        y = jnp.maximum(y + bo_ref[...], 0.0)
        part = jnp.sum(y.reshape(tblk, _N_MOL, _H), axis=0) * (1.0 / _APM)

        @pl.when(i == 0)
        def _init():
            o_ref[...] = part

        @pl.when(i > 0)
        def _acc():
            o_ref[...] = o_ref[...] + part

    xspec = pl.BlockSpec((rblk, _H), lambda i: (i, 0))
    wspec = pl.BlockSpec((_H, _H), lambda i: (0, 0))
    return pl.pallas_call(
        body,
        grid=(grid,),
        in_specs=[xspec, xspec, wspec, wspec,
                  pl.BlockSpec((1, _H), lambda i: (0, 0))],
        out_specs=pl.BlockSpec((_N_MOL, _H), lambda i: (0, 0)),
        out_shape=jax.ShapeDtypeStruct((_N_MOL, _H), jnp.float32),
    )(xf, xb, wof, wob, bo)


def kernel(f_atoms, f_bonds, a2b, b2a, b2revb, W_i_atom, W_i_bond, W_h_0,
           W_h_1, W_lr, W_o, b_o, gru_bias, gru_w_ih_f, gru_w_hh_f,
           gru_b_ih_f, gru_b_hh_f, gru_w_ih_b, gru_w_hh_b, gru_b_ih_b,
           gru_b_hh_b):
    i32 = jnp.int32
    pad_a = jnp.arange((_A_PAD - _NA) * _MAXNB, dtype=i32) * 37 % _NB
    a2b_flat = jnp.concatenate([a2b.astype(i32).reshape(-1), pad_a])
    pad_ba = jnp.arange(_B_PAD - _NB, dtype=i32) * 2 % _NA
    pad_br = jnp.arange(_B_PAD - _NB, dtype=i32) * 37 % _NB
    b2a_p = jnp.concatenate([b2a.astype(i32), pad_ba])
    b2revb_p = jnp.concatenate([b2revb.astype(i32), pad_br])

    grid_b = pl.cdiv(_NB, _BM) * _BM

    input_atom = _tc_mm(f_atoms, W_i_atom, relu=True,
                        out_rows=_A_PAD, grid_rows=_A_PAD)
    input_bond = _tc_mm(f_bonds, W_i_bond, relu=True,
                        out_rows=_B_PAD, grid_rows=grid_b)

    message_atom = input_atom
    message_bond = input_bond
    for W_h in (W_h_0, W_h_1):
        message_atom = _sc_agg(message_bond, a2b_flat, message_atom)
        tmp = _sc_bond(message_atom, message_bond, b2a_p, b2revb_p)
        message_bond = _tc_mm(tmp, W_h, add=input_bond, relu=True,
                              out_rows=_B_PAD, grid_rows=grid_b)

    agg = _sc_agg(message_bond, a2b_flat, None)
    node = _tc_mm3(agg, W_lr[:_H], message_atom, W_lr[_H:2 * _H],
                   input_atom, W_lr[2 * _H:])

    node_tm = node[1:_NA].reshape(_N_MOL, _APM, _H).transpose(1, 0, 2)
    h0 = _tc_h0(node_tm)
    gi_f, gi_b = _tc_gi(
        node_tm.reshape(_N_MOL * _APM, _H), gru_bias.reshape(1, _H),
        gru_w_ih_f.T, gru_b_ih_f.reshape(1, 3 * _H),
        gru_w_ih_b.T, gru_b_ih_b.reshape(1, 3 * _H))
    out_f, out_b = _tc_gru(
        gi_f.reshape(_APM, _N_MOL, 3 * _H), gi_b.reshape(_APM, _N_MOL, 3 * _H),
        h0, gru_w_hh_f.T, gru_w_hh_b.T,
        gru_b_hh_f.reshape(1, 3 * _H), gru_b_hh_b.reshape(1, 3 * _H))

    return _tc_final(out_f.reshape(-1, _H), out_b.reshape(-1, _H),
                     W_o[:_H], W_o[_H:], b_o.reshape(1, _H))


# speedup vs baseline: 5.4833x; 1.0198x over previous
import functools

import jax
import jax.numpy as jnp
from jax import lax
from jax.experimental import pallas as pl
from jax.experimental.pallas import tpu as pltpu
from jax.experimental.pallas import tpu_sc as plsc

_N_MOL = 16
_APM = 625
_NA = 1 + _N_MOL * _APM
_MAXNB = 16
_NB = 1 + _N_MOL * _APM * _MAXNB
_H = 256

_NW = 32
_LANES = 16

_A_CHUNK = 8
_A_CHUNKS_PW = 40
_A_PW = _A_CHUNK * _A_CHUNKS_PW
_A_PAD = _NW * _A_PW

_B_CHUNK = 64
_B_CHUNKS_PW = 80
_B_PW = _B_CHUNK * _B_CHUNKS_PW
_B_PAD = _NW * _B_PW

_BM = 4096


def _sc_mesh():
    return plsc.VectorSubcoreMesh(core_axis_name="c", subcore_axis_name="s")


def _sc_agg(msg_bond, a2b_flat, base):
    add_base = base is not None
    n_idx = _A_CHUNK * _MAXNB
    n_ci = _A_CHUNKS_PW

    def body(*refs):
        if add_base:
            (msg_ref, idx_ref, base_ref, out_ref, idx_all, rows_v, base_v,
             out_v, sg0, sg1, sb0, sb1, so0, so1) = refs
        else:
            (msg_ref, idx_ref, out_ref, idx_all, rows_v,
             out_v, sg0, sg1, sb0, sb1, so0, so1) = refs
        sg = (sg0, sg1)
        sb = (sb0, sb1)
        so = (so0, so1)
        nc = lax.axis_size("c")
        wid = lax.axis_index("s") * nc + lax.axis_index("c")
        wbase = wid * _A_PW

        pltpu.sync_copy(idx_ref.at[pl.ds(wbase * _MAXNB, _A_PW * _MAXNB)],
                        idx_all)

        def start(ci, b):
            pltpu.async_copy(
                msg_ref.at[idx_all.at[pl.ds(ci * n_idx, n_idx)]],
                rows_v.at[b], sg[b])
            if add_base:
                pltpu.async_copy(
                    base_ref.at[pl.ds(wbase + ci * _A_CHUNK, _A_CHUNK)],
                    base_v.at[b], sb[b])

        def wait_in(b):
            pltpu.make_async_copy(msg_ref.at[pl.ds(0, n_idx)],
                                  rows_v.at[b], sg[b]).wait()
            if add_base:
                pltpu.make_async_copy(base_ref.at[pl.ds(0, _A_CHUNK)],
                                      base_v.at[b], sb[b]).wait()

        def compute(ci, b):
            def per_atom(a, _):
                r0 = a * _MAXNB
                for c in range(_H // _LANES):
                    sl = pl.ds(c * _LANES, _LANES)
                    v = [rows_v[b, r0 + j, sl] for j in range(_MAXNB)]
                    m = v[0]
                    for j in range(1, _MAXNB):
                        m = jnp.maximum(m, v[j])
                    n = _MAXNB
                    while n > 1:
                        h = n // 2
                        v = [v[j] + v[j + h] for j in range(h)]
                        n = h
                    res = v[0] * m
                    if add_base:
                        res = base_v[b, a, sl] + res
                    out_v[b, a, sl] = res
                return _

            lax.fori_loop(0, _A_CHUNK, per_atom, None)

        for b in range(2):
            start(b, b)

        def pair(p, _):
            for b in range(2):
                ci = p * 2 + b
                wait_in(b)

                @pl.when(p > 0)
                def _drain_out():
                    pltpu.make_async_copy(
                        out_v.at[b], out_ref.at[pl.ds(0, _A_CHUNK)],
                        so[b]).wait()

                compute(ci, b)
                pltpu.async_copy(out_v.at[b],
                                 out_ref.at[pl.ds(wbase + ci * _A_CHUNK,
                                                  _A_CHUNK)], so[b])

                @pl.when(p < n_ci // 2 - 1)
                def _prefetch():
                    start(ci + 2, b)
            return _

        lax.fori_loop(0, n_ci // 2, pair, None)
        for b in range(2):
            pltpu.make_async_copy(out_v.at[b], out_ref.at[pl.ds(0, _A_CHUNK)],
                                  so[b]).wait()

    scratch = [
        pltpu.VMEM((_A_PW * _MAXNB,), jnp.int32),
        pltpu.VMEM((2, n_idx, _H), jnp.float32),
    ]
    if add_base:
        scratch.append(pltpu.VMEM((2, _A_CHUNK, _H), jnp.float32))
    scratch += [
        pltpu.VMEM((2, _A_CHUNK, _H), jnp.float32),
    ] + [pltpu.SemaphoreType.DMA] * 6
    k = pl.kernel(
        body,
        out_type=jax.ShapeDtypeStruct((_A_PAD, _H), jnp.float32),
        mesh=_sc_mesh(),
        scratch_types=scratch,
    )
    if add_base:
        return k(msg_bond, a2b_flat, base)
    return k(msg_bond, a2b_flat)


def _sc_bond(msg_atom, msg_bond, b2a, b2revb):
    n_ci = _B_CHUNKS_PW

    def body(atom_ref, bond_ref, b2a_ref, b2revb_ref, out_ref,
             idxa_all, idxb_all, rows_a, rows_b, out_v,
             sa0, sa1, sb0, sb1, so0, so1):
        sa = (sa0, sa1)
        sb = (sb0, sb1)
        so = (so0, so1)
        nc = lax.axis_size("c")
        wid = lax.axis_index("s") * nc + lax.axis_index("c")
        wbase = wid * _B_PW

        pltpu.sync_copy(b2a_ref.at[pl.ds(wbase, _B_PW)], idxa_all)
        pltpu.sync_copy(b2revb_ref.at[pl.ds(wbase, _B_PW)], idxb_all)

        def start(ci, b):
            pltpu.async_copy(
                atom_ref.at[idxa_all.at[pl.ds(ci * _B_CHUNK, _B_CHUNK)]],
                rows_a.at[b], sa[b])
            pltpu.async_copy(
                bond_ref.at[idxb_all.at[pl.ds(ci * _B_CHUNK, _B_CHUNK)]],
                rows_b.at[b], sb[b])

        def wait_in(b):
            pltpu.make_async_copy(atom_ref.at[pl.ds(0, _B_CHUNK)],
                                  rows_a.at[b], sa[b]).wait()
            pltpu.make_async_copy(bond_ref.at[pl.ds(0, _B_CHUNK)],
                                  rows_b.at[b], sb[b]).wait()

        def compute(b):
            def per_row(r, _):
                for c in range(_H // _LANES):
                    sl = pl.ds(c * _LANES, _LANES)
                    out_v[b, r, sl] = rows_a[b, r, sl] - rows_b[b, r, sl]
                return _

            lax.fori_loop(0, _B_CHUNK, per_row, None)

        for b in range(2):
            start(b, b)

        def pair(p, _):
            for b in range(2):
                ci = p * 2 + b
                wait_in(b)

                @pl.when(p > 0)
                def _drain_out():
                    pltpu.make_async_copy(
                        out_v.at[b], out_ref.at[pl.ds(0, _B_CHUNK)],
                        so[b]).wait()

                compute(b)
                pltpu.async_copy(out_v.at[b],
                                 out_ref.at[pl.ds(wbase + ci * _B_CHUNK,
                                                  _B_CHUNK)], so[b])

                @pl.when(p < n_ci // 2 - 1)
                def _prefetch():
                    start(ci + 2, b)
            return _

        lax.fori_loop(0, n_ci // 2, pair, None)
        for b in range(2):
            pltpu.make_async_copy(out_v.at[b], out_ref.at[pl.ds(0, _B_CHUNK)],
                                  so[b]).wait()

    return pl.kernel(
        body,
        out_type=jax.ShapeDtypeStruct((_B_PAD, _H), jnp.float32),
        mesh=_sc_mesh(),
        scratch_types=[
            pltpu.VMEM((_B_PW,), jnp.int32),
            pltpu.VMEM((_B_PW,), jnp.int32),
            pltpu.VMEM((2, _B_CHUNK, _H), jnp.float32),
            pltpu.VMEM((2, _B_CHUNK, _H), jnp.float32),
            pltpu.VMEM((2, _B_CHUNK, _H), jnp.float32),
        ] + [pltpu.SemaphoreType.DMA] * 6,
    )(msg_atom, msg_bond, b2a, b2revb)


def _tc_mm(x, w, add=None, relu=False, out_rows=None, grid_rows=None):
    m, kdim = x.shape
    n = w.shape[1]
    out_rows = out_rows or m
    grid_rows = grid_rows or m
    grid = pl.cdiv(grid_rows, _BM)

    def body(*refs):
        if add is not None:
            x_ref, w_ref, a_ref, o_ref = refs
        else:
            x_ref, w_ref, o_ref = refs
        acc = jnp.dot(x_ref[...], w_ref[...], preferred_element_type=jnp.float32)
        if add is not None:
            acc = acc + a_ref[...]
        if relu:
            acc = jnp.maximum(acc, 0.0)
        o_ref[...] = acc

    in_specs = [
        pl.BlockSpec((_BM, kdim), lambda i: (i, 0)),
        pl.BlockSpec((kdim, n), lambda i: (0, 0)),
    ]
    args = [x, w]
    if add is not None:
        in_specs.append(pl.BlockSpec((_BM, n), lambda i: (i, 0)))
        args.append(add)
    return pl.pallas_call(
        body,
        grid=(grid,),
        in_specs=in_specs,
        out_specs=pl.BlockSpec((_BM, n), lambda i: (i, 0)),
        out_shape=jax.ShapeDtypeStruct((out_rows, n), jnp.float32),
    )(*args)


def _tc_mm3(x1, w1, x2, w2, x3, w3):
    grid = _A_PAD // _BM

    def body(x1_ref, w1_ref, x2_ref, w2_ref, x3_ref, w3_ref, o_ref):
        acc = jnp.dot(x1_ref[...], w1_ref[...], preferred_element_type=jnp.float32)
        acc += jnp.dot(x2_ref[...], w2_ref[...], preferred_element_type=jnp.float32)
        acc += jnp.dot(x3_ref[...], w3_ref[...], preferred_element_type=jnp.float32)
        o_ref[...] = acc

    xspec = pl.BlockSpec((_BM, _H), lambda i: (i, 0))
    wspec = pl.BlockSpec((_H, _H), lambda i: (0, 0))
    return pl.pallas_call(
        body,
        grid=(grid,),
        in_specs=[xspec, wspec, xspec, wspec, xspec, wspec],
        out_specs=xspec,
        out_shape=jax.ShapeDtypeStruct((_A_PAD, _H), jnp.float32),
    )(x1, w1, x2, w2, x3, w3)


def _tc_h0(node_tm):
    tblk = 125
    grid = _APM // tblk

    def body(x_ref, o_ref):
        i = pl.program_id(0)
        bm = jnp.max(x_ref[...], axis=0)

        @pl.when(i == 0)
        def _init():
            o_ref[...] = bm

        @pl.when(i > 0)
        def _acc():
            o_ref[...] = jnp.maximum(o_ref[...], bm)

    return pl.pallas_call(
        body,
        grid=(grid,),
        in_specs=[pl.BlockSpec((tblk, _N_MOL, _H), lambda i: (i, 0, 0))],
        out_specs=pl.BlockSpec((_N_MOL, _H), lambda i: (0, 0)),
        out_shape=jax.ShapeDtypeStruct((_N_MOL, _H), jnp.float32),
    )(node_tm)


def _tc_gi(node_flat, gru_bias, wf_t, bf, wb_t, bb):
    m = node_flat.shape[0]
    bm = min(_BM, 2048)
    grid = pl.cdiv(m, bm)

    def body(x_ref, gbias_ref, wf_ref, bf_ref, wb_ref, bb_ref, of_ref, ob_ref):
        msg = jnp.maximum(x_ref[...] + gbias_ref[...], 0.0)
        of_ref[...] = jnp.dot(msg, wf_ref[...], preferred_element_type=jnp.float32) + bf_ref[...]
        ob_ref[...] = jnp.dot(msg, wb_ref[...], preferred_element_type=jnp.float32) + bb_ref[...]

    wspec = pl.BlockSpec((_H, 3 * _H), lambda i: (0, 0))
    bspec = pl.BlockSpec((1, 3 * _H), lambda i: (0, 0))
    ospec = pl.BlockSpec((bm, 3 * _H), lambda i: (i, 0))
    oshape = jax.ShapeDtypeStruct((m, 3 * _H), jnp.float32)
    return pl.pallas_call(
        body,
        grid=(grid,),
        in_specs=[
            pl.BlockSpec((bm, _H), lambda i: (i, 0)),
            pl.BlockSpec((1, _H), lambda i: (0, 0)),
            wspec, bspec, wspec, bspec,
        ],
        out_specs=[ospec, ospec],
        out_shape=[oshape, oshape],
    )(node_flat, gru_bias, wf_t, bf, wb_t, bb)


def _tc_gru(gi_f, gi_b, h0, whhf_t, whhb_t, bhf, bhb):
    T = _APM
    TSUB = 25
    NBLK = T // TSUB

    def body(gif_ref, gib_ref, h0_ref, wf_ref, wb_ref, bf_ref, bb_ref,
             of_ref, ob_ref, hf, hb):
        t = pl.program_id(0)

        @pl.when(t == 0)
        def _init():
            hf[...] = h0_ref[...]
            hb[...] = h0_ref[...]

        def step(gi, h, w_ref, b_ref):
            gh = jnp.dot(h, w_ref[...], preferred_element_type=jnp.float32) + b_ref[...]
            i_r = gi[:, :_H]
            i_z = gi[:, _H:2 * _H]
            i_n = gi[:, 2 * _H:]
            h_r = gh[:, :_H]
            h_z = gh[:, _H:2 * _H]
            h_n = gh[:, 2 * _H:]
            r = jax.nn.sigmoid(i_r + h_r)
            z = jax.nn.sigmoid(i_z + h_z)
            n = jnp.tanh(i_n + r * h_n)
            return (1.0 - z) * n + z * h

        hfv = hf[...]
        hbv = hb[...]
        for k in range(TSUB):
            hfv = step(gif_ref[k], hfv, wf_ref, bf_ref)
            of_ref[k] = hfv
            kb = TSUB - 1 - k
            hbv = step(gib_ref[kb], hbv, wb_ref, bb_ref)
            ob_ref[kb] = hbv
        hf[...] = hfv
        hb[...] = hbv

    gspec_f = pl.BlockSpec((TSUB, _N_MOL, 3 * _H), lambda t: (t, 0, 0))
    gspec_b = pl.BlockSpec((TSUB, _N_MOL, 3 * _H), lambda t: (NBLK - 1 - t, 0, 0))
    wspec = pl.BlockSpec((_H, 3 * _H), lambda t: (0, 0))
    bspec = pl.BlockSpec((1, 3 * _H), lambda t: (0, 0))
    ospec_f = pl.BlockSpec((TSUB, _N_MOL, _H), lambda t: (t, 0, 0))
    ospec_b = pl.BlockSpec((TSUB, _N_MOL, _H), lambda t: (NBLK - 1 - t, 0, 0))
    oshape = jax.ShapeDtypeStruct((T, _N_MOL, _H), jnp.float32)
    return pl.pallas_call(
        body,
        grid=(NBLK,),
        in_specs=[
            gspec_f, gspec_b,
            pl.BlockSpec((_N_MOL, _H), lambda t: (0, 0)),
            wspec, wspec, bspec, bspec,
        ],
        out_specs=[ospec_f, ospec_b],
        out_shape=[oshape, oshape],
        scratch_shapes=[
            pltpu.VMEM((_N_MOL, _H), jnp.float32),
            pltpu.VMEM((_N_MOL, _H), jnp.float32),
        ],
    )(gi_f, gi_b, h0, whhf_t, whhb_t, bhf, bhb)


def _tc_final(xf, xb, wof, wob, bo):
    tblk = 125
    rblk = tblk * _N_MOL
    grid = _APM // tblk

    def body(xf_ref, xb_ref, wof_ref, wob_ref, bo_ref, o_ref):
        i = pl.program_id(0)
        y = jnp.dot(xf_ref[...], wof_ref[...], preferred_element_type=jnp.float32)
        y += jnp.dot(xb_ref[...], wob_ref[...], preferred_element_type=jnp.float32)
        y = jnp.maximum(y + bo_ref[...], 0.0)
        part = jnp.sum(y.reshape(tblk, _N_MOL, _H), axis=0) * (1.0 / _APM)

        @pl.when(i == 0)
        def _init():
            o_ref[...] = part

        @pl.when(i > 0)
        def _acc():
            o_ref[...] = o_ref[...] + part

    xspec = pl.BlockSpec((rblk, _H), lambda i: (i, 0))
    wspec = pl.BlockSpec((_H, _H), lambda i: (0, 0))
    return pl.pallas_call(
        body,
        grid=(grid,),
        in_specs=[xspec, xspec, wspec, wspec,
                  pl.BlockSpec((1, _H), lambda i: (0, 0))],
        out_specs=pl.BlockSpec((_N_MOL, _H), lambda i: (0, 0)),
        out_shape=jax.ShapeDtypeStruct((_N_MOL, _H), jnp.float32),
    )(xf, xb, wof, wob, bo)


def kernel(f_atoms, f_bonds, a2b, b2a, b2revb, W_i_atom, W_i_bond, W_h_0,
           W_h_1, W_lr, W_o, b_o, gru_bias, gru_w_ih_f, gru_w_hh_f,
           gru_b_ih_f, gru_b_hh_f, gru_w_ih_b, gru_w_hh_b, gru_b_ih_b,
           gru_b_hh_b):
    i32 = jnp.int32
    pad_a = jnp.arange((_A_PAD - _NA) * _MAXNB, dtype=i32) * 37 % _NB
    a2b_flat = jnp.concatenate([a2b.astype(i32).reshape(-1), pad_a])
    pad_ba = jnp.arange(_B_PAD - _NB, dtype=i32) * 2 % _NA
    pad_br = jnp.arange(_B_PAD - _NB, dtype=i32) * 37 % _NB
    b2a_p = jnp.concatenate([b2a.astype(i32), pad_ba])
    b2revb_p = jnp.concatenate([b2revb.astype(i32), pad_br])

    grid_b = pl.cdiv(_NB, _BM) * _BM

    input_atom = _tc_mm(f_atoms, W_i_atom, relu=True,
                        out_rows=_A_PAD, grid_rows=_A_PAD)
    input_bond = _tc_mm(f_bonds, W_i_bond, relu=True,
                        out_rows=_B_PAD, grid_rows=grid_b)

    message_atom = input_atom
    message_bond = input_bond
    for W_h in (W_h_0, W_h_1):
        message_atom = _sc_agg(message_bond, a2b_flat, message_atom)
        tmp = _sc_bond(message_atom, message_bond, b2a_p, b2revb_p)
        message_bond = _tc_mm(tmp, W_h, add=input_bond, relu=True,
                              out_rows=_B_PAD, grid_rows=grid_b)

    agg = _sc_agg(message_bond, a2b_flat, None)
    node = _tc_mm3(agg, W_lr[:_H], message_atom, W_lr[_H:2 * _H],
                   input_atom, W_lr[2 * _H:])

    node_tm = node[1:_NA].reshape(_N_MOL, _APM, _H).transpose(1, 0, 2)
    h0 = _tc_h0(node_tm)
    gi_f, gi_b = _tc_gi(
        node_tm.reshape(_N_MOL * _APM, _H), gru_bias.reshape(1, _H),
        gru_w_ih_f.T, gru_b_ih_f.reshape(1, 3 * _H),
        gru_w_ih_b.T, gru_b_ih_b.reshape(1, 3 * _H))
    out_f, out_b = _tc_gru(
        gi_f.reshape(_APM, _N_MOL, 3 * _H), gi_b.reshape(_APM, _N_MOL, 3 * _H),
        h0, gru_w_hh_f.T, gru_w_hh_b.T,
        gru_b_hh_f.reshape(1, 3 * _H), gru_b_hh_b.reshape(1, 3 * _H))

    return _tc_final(out_f.reshape(-1, _H), out_b.reshape(-1, _H),
                     W_o[:_H], W_o[_H:], b_o.reshape(1, _H))
